# Initial kernel scaffold; baseline (speedup 1.0000x reference)
import jax
import jax.numpy as jnp
from jax.experimental import pallas as pl


def kernel(x, edge_index, batch, W1, a1_src, a1_dst, b1, W2, a2_src, a2_dst, b2, W3, a3_src, a3_dst, b3, W_news, b_news, W_l1, b_l1, W_l2, b_l2):
    raise NotImplementedError("write your pallas kernel here")


# speedup vs baseline: 19.4862x; 19.4862x over previous
import dataclasses
import functools

import jax
import jax.numpy as jnp
from jax import lax
from jax.experimental import pallas as pl
from jax.experimental.pallas import tpu as pltpu
from jax.experimental.pallas import tpu_sc as plsc

N = 10000
D = 128
G = 128
NPAD = 10240
NTILES = 32
EPAD = 331776
EPT = EPAD // 16
DH = 64
DQ = 32
CH = 128
RPT = NPAD // 16
ZR = 64
BLK = 1280

def _sc_edge_body(src_hbm, dst_hbm, sad_hbm, h_hbm, den_hbm, acc_hbm,
                  src_v, dst_v, as_v, ad_v, den_v, ex_v, rowb, dbuf, gbuf,
                  zbuf, acc_sh):
    cid = lax.axis_index("c")
    sid = lax.axis_index("s")
    base = sid * EPT
    pltpu.sync_copy(src_hbm.at[pl.ds(base, EPT)], src_v)
    pltpu.sync_copy(dst_hbm.at[pl.ds(base, EPT)], dst_v)
    pltpu.sync_copy(sad_hbm.at[0], as_v)
    pltpu.sync_copy(sad_hbm.at[1], ad_v)

    zero16 = jnp.zeros((16,), jnp.float32)

    @pl.loop(0, NPAD, step=16)
    def _(i):
        den_v[pl.ds(i, 16)] = zero16

    @pl.loop(0, ZR)
    def _(r):
        for c in range(DQ // 16):
            zbuf[r, pl.ds(c * 16, 16)] = zero16

    @pl.loop(0, EPT, step=16)
    def _(i):
        s16 = src_v[pl.ds(i, 16)]
        d16 = dst_v[pl.ds(i, 16)]
        z = plsc.load_gather(as_v, [s16]) + plsc.load_gather(ad_v, [d16])
        e = jnp.maximum(z, 0.2 * z)
        ex = jnp.exp(e)
        ex_v[pl.ds(i, 16)] = ex
        plsc.addupdate_scatter(den_v, [d16], ex)

    @pl.when(cid == 0)
    def _():
        pltpu.sync_copy(den_v, den_hbm.at[sid])

    for ph in range(2):
        q = cid * 2 + ph
        row_off = q * NPAD

        @pl.loop(0, RPT, step=ZR)
        def _(r):
            pltpu.sync_copy(zbuf, acc_sh.at[pl.ds(sid * RPT + r, ZR)])

        plsc.subcore_barrier()

        @pl.loop(0, EPT, step=CH)
        def _(i):
            for k in range(0, CH, 16):
                dbuf[pl.ds(k, 16)] = dst_v[pl.ds(i + k, 16)]
                gbuf[pl.ds(k, 16)] = src_v[pl.ds(i + k, 16)] + row_off
            pltpu.sync_copy(h_hbm.at[gbuf], rowb)

            @pl.loop(0, CH, step=16)
            def _(k):
                wv = ex_v[pl.ds(i + k, 16)]
                for e in range(16):
                    w = wv[e]
                    for c in range(DQ // 16):
                        sl = pl.ds(c * 16, 16)
                        rowb[k + e, sl] = rowb[k + e, sl] * w

            pltpu.sync_copy(rowb, acc_sh.at[dbuf], add=True)

        plsc.subcore_barrier()

        @pl.loop(0, RPT, step=ZR)
        def _(r):
            pltpu.sync_copy(acc_sh.at[pl.ds(sid * RPT + r, ZR)],
                            acc_hbm.at[q].at[pl.ds(sid * RPT + r, ZR)])


def _sc_compiler_params():
    cp = pltpu.CompilerParams()
    fields = pltpu.CompilerParams.__dataclass_fields__
    if "needs_layout_passes" in fields:
        cp = dataclasses.replace(cp, needs_layout_passes=False)
    if "use_tc_tiling_on_sc" in fields:
        cp = dataclasses.replace(cp, use_tc_tiling_on_sc=False)
    return cp


def _sc_edge(src, dst, sad, h):
    k = pl.kernel(
        _sc_edge_body,
        out_type=(jax.ShapeDtypeStruct((16, NPAD), jnp.float32),
                  jax.ShapeDtypeStruct((4, NPAD, DQ), jnp.float32)),
        mesh=plsc.VectorSubcoreMesh(core_axis_name="c", subcore_axis_name="s"),
        scratch_types=[
            pltpu.VMEM((EPT,), jnp.int32),
            pltpu.VMEM((EPT,), jnp.int32),
            pltpu.VMEM((NPAD,), jnp.float32),
            pltpu.VMEM((NPAD,), jnp.float32),
            pltpu.VMEM((NPAD,), jnp.float32),
            pltpu.VMEM((EPT,), jnp.float32),
            pltpu.VMEM((CH, DQ), jnp.float32),
            pltpu.VMEM((CH,), jnp.int32),
            pltpu.VMEM((CH,), jnp.int32),
            pltpu.VMEM((ZR, DQ), jnp.float32),
            pltpu.VMEM_SHARED((NPAD, DQ), jnp.float32),
        ],
        compiler_params=_sc_compiler_params(),
    )
    return k(src, dst, sad, h)


def _tc_proj1_body(x_ref, w_ref, ap_ref, h_ref, sad_ref):
    h = jnp.dot(x_ref[...], w_ref[...], preferred_element_type=jnp.float32)
    h_ref[...] = h
    sad_ref[...] = lax.dot_general(ap_ref[...], h, (((0,), (1,)), ((), ())),
                                   preferred_element_type=jnp.float32)


def _tc_proj1(x_pad, w, ap):
    return pl.pallas_call(
        _tc_proj1_body,
        grid=(NPAD // BLK,),
        in_specs=[
            pl.BlockSpec((BLK, D), lambda i: (i, 0)),
            pl.BlockSpec((D, D), lambda i: (0, 0)),
            pl.BlockSpec((D, 2), lambda i: (0, 0)),
        ],
        out_specs=[
            pl.BlockSpec((BLK, D), lambda i: (i, 0)),
            pl.BlockSpec((2, BLK), lambda i: (0, i)),
        ],
        out_shape=[
            jax.ShapeDtypeStruct((NPAD, D), jnp.float32),
            jax.ShapeDtypeStruct((2, NPAD), jnp.float32),
        ],
    )(x_pad, w, ap)


def _tc_comb_body(acc_ref, den_ref, b_ref, w_ref, ap_ref, h_ref, sad_ref):
    i = pl.program_id(0)
    den = jnp.sum(den_ref[...], axis=0)
    invd = 1.0 / (den + 1e-16)
    rows = lax.broadcasted_iota(jnp.int32, (BLK, 1), 0) + i * BLK
    acc = jnp.concatenate([acc_ref[q] for q in range(4)], axis=1)
    hin = acc * invd[:, None] + b_ref[...]
    hin = jnp.where(rows < N, jnp.maximum(hin, 0.0), 0.0)
    h = jnp.dot(hin, w_ref[...], preferred_element_type=jnp.float32)
    h_ref[...] = h
    sad_ref[...] = lax.dot_general(ap_ref[...], h, (((0,), (1,)), ((), ())),
                                   preferred_element_type=jnp.float32)


def _tc_comb(acc, den, b, w, ap):
    return pl.pallas_call(
        _tc_comb_body,
        grid=(NPAD // BLK,),
        in_specs=[
            pl.BlockSpec((4, BLK, DQ), lambda i: (0, i, 0)),
            pl.BlockSpec((16, BLK), lambda i: (0, i)),
            pl.BlockSpec((1, D), lambda i: (0, 0)),
            pl.BlockSpec((D, D), lambda i: (0, 0)),
            pl.BlockSpec((D, 2), lambda i: (0, 0)),
        ],
        out_specs=[
            pl.BlockSpec((BLK, D), lambda i: (i, 0)),
            pl.BlockSpec((2, BLK), lambda i: (0, i)),
        ],
        out_shape=[
            jax.ShapeDtypeStruct((NPAD, D), jnp.float32),
            jax.ShapeDtypeStruct((2, NPAD), jnp.float32),
        ],
    )(acc, den, b, w, ap)


def _tc_final_body(acc_ref, den_ref, b3_ref, x_ref, batch_ref,
                   wl1_ref, bl1_ref, wn_ref, bn_ref, wa_ref, wb_ref, bl2_ref,
                   o_ref, h3_ref, tbl_ref, xr_ref, rt_ref):
    den = jnp.sum(den_ref[...], axis=0)
    invd = 1.0 / (den + 1e-16)
    rows = lax.broadcasted_iota(jnp.int32, (NPAD, 1), 0)
    acc = jnp.concatenate([acc_ref[q] for q in range(4)], axis=1)
    h3 = acc * invd[:, None] + b3_ref[...]
    h3_ref[...] = jnp.where(rows < N, jnp.maximum(h3, 0.0), 0.0)
    tbl_ref[...] = jnp.full((G, D), -jnp.inf, jnp.float32)

    def init_rt(g, carry):
        rt_ref[g] = jnp.int32(2147483647)
        return carry

    lax.fori_loop(0, G, init_rt, 0)

    def pool_body(i, carry):
        g = batch_ref[i]
        row = h3_ref[pl.ds(i, 1), :]
        cur = tbl_ref[pl.ds(g, 1), :]
        tbl_ref[pl.ds(g, 1), :] = jnp.maximum(cur, row)
        rt_ref[g] = jnp.minimum(rt_ref[g], i)
        return carry

    lax.fori_loop(0, N, pool_body, 0)

    pooled = tbl_ref[...]
    pooled = jnp.where(jnp.isfinite(pooled), pooled, 0.0)
    gm = jnp.maximum(
        jnp.dot(pooled, wl1_ref[...], preferred_element_type=jnp.float32)
        + bl1_ref[...], 0.0)

    def root_body(g, carry):
        idx = jnp.minimum(rt_ref[g], N - 1)
        xr_ref[pl.ds(g, 1), :] = x_ref[pl.ds(idx, 1), :]
        return carry

    lax.fori_loop(0, G, root_body, 0)

    news = jnp.maximum(
        jnp.dot(xr_ref[...], wn_ref[...], preferred_element_type=jnp.float32)
        + bn_ref[...], 0.0)
    logit = (jnp.dot(gm, wa_ref[...], preferred_element_type=jnp.float32)
             + jnp.dot(news, wb_ref[...], preferred_element_type=jnp.float32)
             + bl2_ref[...])
    o_ref[...] = jax.nn.sigmoid(logit)


def _tc_final(acc, den, b3, x_pad, batch, wl1, bl1, wn, bn, wa, wb, bl2):
    return pl.pallas_call(
        _tc_final_body,
        in_specs=[
            pl.BlockSpec((4, NPAD, DQ), lambda: (0, 0, 0)),
            pl.BlockSpec((16, NPAD), lambda: (0, 0)),
            pl.BlockSpec((1, D), lambda: (0, 0)),
            pl.BlockSpec((NPAD, D), lambda: (0, 0)),
            pl.BlockSpec(memory_space=pltpu.SMEM),
            pl.BlockSpec((D, D), lambda: (0, 0)),
            pl.BlockSpec((1, D), lambda: (0, 0)),
            pl.BlockSpec((D, D), lambda: (0, 0)),
            pl.BlockSpec((1, D), lambda: (0, 0)),
            pl.BlockSpec((D, 1), lambda: (0, 0)),
            pl.BlockSpec((D, 1), lambda: (0, 0)),
            pl.BlockSpec((1, 1), lambda: (0, 0)),
        ],
        out_specs=pl.BlockSpec((G, 1), lambda: (0, 0)),
        out_shape=jax.ShapeDtypeStruct((G, 1), jnp.float32),
        scratch_shapes=[
            pltpu.VMEM((NPAD, D), jnp.float32),
            pltpu.VMEM((G, D), jnp.float32),
            pltpu.VMEM((G, D), jnp.float32),
            pltpu.SMEM((G,), jnp.int32),
        ],
    )(acc, den, b3, x_pad, batch, wl1, bl1, wn, bn, wa, wb, bl2)


def kernel(x, edge_index, batch, W1, a1_src, a1_dst, b1, W2, a2_src, a2_dst,
           b2, W3, a3_src, a3_dst, b3, W_news, b_news, W_l1, b_l1, W_l2,
           b_l2):
    e_real = edge_index.shape[1] + N
    npad_e = EPAD - e_real
    loops = jnp.arange(N, dtype=jnp.int32)
    src = jnp.concatenate([edge_index[0].astype(jnp.int32), loops,
                           jnp.zeros((npad_e,), jnp.int32)])
    dst = jnp.concatenate([edge_index[1].astype(jnp.int32), loops,
                           jnp.full((npad_e,), N, jnp.int32)])
    x_pad = jnp.pad(x, ((0, NPAD - N), (0, 0)))

    def split(h):
        return jnp.concatenate([h[:, q * DQ:(q + 1) * DQ] for q in range(4)],
                               axis=0)

    h1, sad1 = _tc_proj1(x_pad, W1, jnp.stack([a1_src, a1_dst], axis=1))
    den1, acc1 = _sc_edge(src, dst, sad1, split(h1))

    h2, sad2 = _tc_comb(acc1, den1, b1.reshape(1, D), W2,
                        jnp.stack([a2_src, a2_dst], axis=1))
    den2, acc2 = _sc_edge(src, dst, sad2, split(h2))

    h3, sad3 = _tc_comb(acc2, den2, b2.reshape(1, D), W3,
                        jnp.stack([a3_src, a3_dst], axis=1))
    den3, acc3 = _sc_edge(src, dst, sad3, split(h3))

    return _tc_final(acc3, den3, b3.reshape(1, D), x_pad, batch,
                     W_l1, b_l1.reshape(1, D), W_news, b_news.reshape(1, D),
                     W_l2[:D], W_l2[D:], b_l2.reshape(1, 1))


# speedup vs baseline: 28.2580x; 1.4502x over previous
import dataclasses
import functools

import jax
import jax.numpy as jnp
from jax import lax
from jax.experimental import pallas as pl
from jax.experimental.pallas import tpu as pltpu
from jax.experimental.pallas import tpu_sc as plsc

N = 10000
D = 128
G = 128
NPAD = 10240
NTILES = 32
EPAD = 331776
EPT = EPAD // 16
DH = 64
DQ = 32
CH = 128
RPT = NPAD // 16
ZR = 64
BLK = 1280

def _sc_edge_body(src_hbm, dst_hbm, sad_hbm, h_hbm, den_hbm, acc_hbm,
                  src_v, dst_v, as_v, ad_v, den_v, ex_v, rowb0, rowb1,
                  dbuf0, dbuf1, gbuf0, gbuf1, zbuf, acc_sh, sem0, sem1):
    cid = lax.axis_index("c")
    sid = lax.axis_index("s")
    base = sid * EPT
    pltpu.sync_copy(src_hbm.at[pl.ds(base, EPT)], src_v)
    pltpu.sync_copy(dst_hbm.at[pl.ds(base, EPT)], dst_v)
    pltpu.sync_copy(sad_hbm.at[0], as_v)
    pltpu.sync_copy(sad_hbm.at[1], ad_v)

    zero16 = jnp.zeros((16,), jnp.float32)

    @pl.loop(0, NPAD, step=16)
    def _(i):
        den_v[pl.ds(i, 16)] = zero16

    @pl.loop(0, ZR)
    def _(r):
        for c in range(DQ // 16):
            zbuf[r, pl.ds(c * 16, 16)] = zero16

    @pl.loop(0, EPT, step=16)
    def _(i):
        s16 = src_v[pl.ds(i, 16)]
        d16 = dst_v[pl.ds(i, 16)]
        z = plsc.load_gather(as_v, [s16]) + plsc.load_gather(ad_v, [d16])
        e = jnp.maximum(z, 0.2 * z)
        ex = jnp.exp(e)
        ex_v[pl.ds(i, 16)] = ex
        plsc.addupdate_scatter(den_v, [d16], ex)

    @pl.when(cid == 0)
    def _():
        pltpu.sync_copy(den_v, den_hbm.at[sid])

    rowbs = (rowb0, rowb1)
    dbufs = (dbuf0, dbuf1)
    gbufs = (gbuf0, gbuf1)
    sems = (sem0, sem1)

    for ph in range(2):
        q = cid * 2 + ph
        row_off = q * NPAD

        @pl.loop(0, RPT, step=ZR)
        def _(r):
            pltpu.sync_copy(zbuf, acc_sh.at[pl.ds(sid * RPT + r, ZR)])

        plsc.subcore_barrier()

        def prep_and_start(i, p):
            for k in range(0, CH, 16):
                dbufs[p][pl.ds(k, 16)] = dst_v[pl.ds(i + k, 16)]
                gbufs[p][pl.ds(k, 16)] = src_v[pl.ds(i + k, 16)] + row_off
            pltpu.async_copy(h_hbm.at[gbufs[p]], rowbs[p], sems[p])

        def finish(i, p):
            pltpu.make_async_copy(h_hbm.at[gbufs[p]], rowbs[p],
                                  sems[p]).wait()

            @pl.loop(0, CH, step=16)
            def _(k):
                wv = ex_v[pl.ds(i + k, 16)]
                for e in range(16):
                    w = wv[e]
                    for c in range(DQ // 16):
                        sl = pl.ds(c * 16, 16)
                        rowbs[p][k + e, sl] = rowbs[p][k + e, sl] * w

            pltpu.sync_copy(rowbs[p], acc_sh.at[dbufs[p]], add=True)

        prep_and_start(0, 0)

        @pl.loop(0, EPT, step=2 * CH)
        def _(i):
            prep_and_start(i + CH, 1)
            finish(i, 0)

            @pl.when(i + 2 * CH < EPT)
            def _():
                prep_and_start(i + 2 * CH, 0)

            finish(i + CH, 1)

        plsc.subcore_barrier()

        @pl.loop(0, RPT, step=ZR)
        def _(r):
            pltpu.sync_copy(acc_sh.at[pl.ds(sid * RPT + r, ZR)],
                            acc_hbm.at[q].at[pl.ds(sid * RPT + r, ZR)])


def _sc_compiler_params():
    cp = pltpu.CompilerParams()
    fields = pltpu.CompilerParams.__dataclass_fields__
    if "needs_layout_passes" in fields:
        cp = dataclasses.replace(cp, needs_layout_passes=False)
    if "use_tc_tiling_on_sc" in fields:
        cp = dataclasses.replace(cp, use_tc_tiling_on_sc=False)
    return cp


def _sc_edge(src, dst, sad, h):
    k = pl.kernel(
        _sc_edge_body,
        out_type=(jax.ShapeDtypeStruct((16, NPAD), jnp.float32),
                  jax.ShapeDtypeStruct((4, NPAD, DQ), jnp.float32)),
        mesh=plsc.VectorSubcoreMesh(core_axis_name="c", subcore_axis_name="s"),
        scratch_types=[
            pltpu.VMEM((EPT,), jnp.int32),
            pltpu.VMEM((EPT,), jnp.int32),
            pltpu.VMEM((NPAD,), jnp.float32),
            pltpu.VMEM((NPAD,), jnp.float32),
            pltpu.VMEM((NPAD,), jnp.float32),
            pltpu.VMEM((EPT,), jnp.float32),
            pltpu.VMEM((CH, DQ), jnp.float32),
            pltpu.VMEM((CH, DQ), jnp.float32),
            pltpu.VMEM((CH,), jnp.int32),
            pltpu.VMEM((CH,), jnp.int32),
            pltpu.VMEM((CH,), jnp.int32),
            pltpu.VMEM((CH,), jnp.int32),
            pltpu.VMEM((ZR, DQ), jnp.float32),
            pltpu.VMEM_SHARED((NPAD, DQ), jnp.float32),
            pltpu.SemaphoreType.DMA,
            pltpu.SemaphoreType.DMA,
        ],
        compiler_params=_sc_compiler_params(),
    )
    return k(src, dst, sad, h)


def _tc_proj1_body(x_ref, w_ref, ap_ref, h_ref, sad_ref):
    h = jnp.dot(x_ref[...], w_ref[...], preferred_element_type=jnp.float32)
    h_ref[...] = h
    sad_ref[...] = lax.dot_general(ap_ref[...], h, (((0,), (1,)), ((), ())),
                                   preferred_element_type=jnp.float32)


def _tc_proj1(x_pad, w, ap):
    return pl.pallas_call(
        _tc_proj1_body,
        grid=(NPAD // BLK,),
        in_specs=[
            pl.BlockSpec((BLK, D), lambda i: (i, 0)),
            pl.BlockSpec((D, D), lambda i: (0, 0)),
            pl.BlockSpec((D, 2), lambda i: (0, 0)),
        ],
        out_specs=[
            pl.BlockSpec((BLK, D), lambda i: (i, 0)),
            pl.BlockSpec((2, BLK), lambda i: (0, i)),
        ],
        out_shape=[
            jax.ShapeDtypeStruct((NPAD, D), jnp.float32),
            jax.ShapeDtypeStruct((2, NPAD), jnp.float32),
        ],
    )(x_pad, w, ap)


def _tc_comb_body(acc_ref, den_ref, b_ref, w_ref, ap_ref, h_ref, sad_ref):
    i = pl.program_id(0)
    den = jnp.sum(den_ref[...], axis=0)
    invd = 1.0 / (den + 1e-16)
    rows = lax.broadcasted_iota(jnp.int32, (BLK, 1), 0) + i * BLK
    acc = jnp.concatenate([acc_ref[q] for q in range(4)], axis=1)
    hin = acc * invd[:, None] + b_ref[...]
    hin = jnp.where(rows < N, jnp.maximum(hin, 0.0), 0.0)
    h = jnp.dot(hin, w_ref[...], preferred_element_type=jnp.float32)
    h_ref[...] = h
    sad_ref[...] = lax.dot_general(ap_ref[...], h, (((0,), (1,)), ((), ())),
                                   preferred_element_type=jnp.float32)


def _tc_comb(acc, den, b, w, ap):
    return pl.pallas_call(
        _tc_comb_body,
        grid=(NPAD // BLK,),
        in_specs=[
            pl.BlockSpec((4, BLK, DQ), lambda i: (0, i, 0)),
            pl.BlockSpec((16, BLK), lambda i: (0, i)),
            pl.BlockSpec((1, D), lambda i: (0, 0)),
            pl.BlockSpec((D, D), lambda i: (0, 0)),
            pl.BlockSpec((D, 2), lambda i: (0, 0)),
        ],
        out_specs=[
            pl.BlockSpec((BLK, D), lambda i: (i, 0)),
            pl.BlockSpec((2, BLK), lambda i: (0, i)),
        ],
        out_shape=[
            jax.ShapeDtypeStruct((NPAD, D), jnp.float32),
            jax.ShapeDtypeStruct((2, NPAD), jnp.float32),
        ],
    )(acc, den, b, w, ap)


def _tc_final_body(acc_ref, den_ref, b3_ref, x_ref, batch_ref,
                   wl1_ref, bl1_ref, wn_ref, bn_ref, wa_ref, wb_ref, bl2_ref,
                   o_ref, h3_ref, tbl_ref, xr_ref, rt_ref):
    den = jnp.sum(den_ref[...], axis=0)
    invd = 1.0 / (den + 1e-16)
    rows = lax.broadcasted_iota(jnp.int32, (NPAD, 1), 0)
    acc = jnp.concatenate([acc_ref[q] for q in range(4)], axis=1)
    h3 = acc * invd[:, None] + b3_ref[...]
    h3_ref[...] = jnp.where(rows < N, jnp.maximum(h3, 0.0), 0.0)
    tbl_ref[...] = jnp.full((G, D), -jnp.inf, jnp.float32)

    def init_rt(g, carry):
        rt_ref[g] = jnp.int32(2147483647)
        return carry

    lax.fori_loop(0, G, init_rt, 0)

    def pool_body(i, carry):
        g = batch_ref[i]
        row = h3_ref[pl.ds(i, 1), :]
        cur = tbl_ref[pl.ds(g, 1), :]
        tbl_ref[pl.ds(g, 1), :] = jnp.maximum(cur, row)
        rt_ref[g] = jnp.minimum(rt_ref[g], i)
        return carry

    lax.fori_loop(0, N, pool_body, 0)

    pooled = tbl_ref[...]
    pooled = jnp.where(jnp.isfinite(pooled), pooled, 0.0)
    gm = jnp.maximum(
        jnp.dot(pooled, wl1_ref[...], preferred_element_type=jnp.float32)
        + bl1_ref[...], 0.0)

    def root_body(g, carry):
        idx = jnp.minimum(rt_ref[g], N - 1)
        xr_ref[pl.ds(g, 1), :] = x_ref[pl.ds(idx, 1), :]
        return carry

    lax.fori_loop(0, G, root_body, 0)

    news = jnp.maximum(
        jnp.dot(xr_ref[...], wn_ref[...], preferred_element_type=jnp.float32)
        + bn_ref[...], 0.0)
    logit = (jnp.dot(gm, wa_ref[...], preferred_element_type=jnp.float32)
             + jnp.dot(news, wb_ref[...], preferred_element_type=jnp.float32)
             + bl2_ref[...])
    o_ref[...] = jax.nn.sigmoid(logit)


def _tc_final(acc, den, b3, x_pad, batch, wl1, bl1, wn, bn, wa, wb, bl2):
    return pl.pallas_call(
        _tc_final_body,
        in_specs=[
            pl.BlockSpec((4, NPAD, DQ), lambda: (0, 0, 0)),
            pl.BlockSpec((16, NPAD), lambda: (0, 0)),
            pl.BlockSpec((1, D), lambda: (0, 0)),
            pl.BlockSpec((NPAD, D), lambda: (0, 0)),
            pl.BlockSpec(memory_space=pltpu.SMEM),
            pl.BlockSpec((D, D), lambda: (0, 0)),
            pl.BlockSpec((1, D), lambda: (0, 0)),
            pl.BlockSpec((D, D), lambda: (0, 0)),
            pl.BlockSpec((1, D), lambda: (0, 0)),
            pl.BlockSpec((D, 1), lambda: (0, 0)),
            pl.BlockSpec((D, 1), lambda: (0, 0)),
            pl.BlockSpec((1, 1), lambda: (0, 0)),
        ],
        out_specs=pl.BlockSpec((G, 1), lambda: (0, 0)),
        out_shape=jax.ShapeDtypeStruct((G, 1), jnp.float32),
        scratch_shapes=[
            pltpu.VMEM((NPAD, D), jnp.float32),
            pltpu.VMEM((G, D), jnp.float32),
            pltpu.VMEM((G, D), jnp.float32),
            pltpu.SMEM((G,), jnp.int32),
        ],
    )(acc, den, b3, x_pad, batch, wl1, bl1, wn, bn, wa, wb, bl2)


def kernel(x, edge_index, batch, W1, a1_src, a1_dst, b1, W2, a2_src, a2_dst,
           b2, W3, a3_src, a3_dst, b3, W_news, b_news, W_l1, b_l1, W_l2,
           b_l2):
    e_real = edge_index.shape[1] + N
    npad_e = EPAD - e_real
    loops = jnp.arange(N, dtype=jnp.int32)
    src = jnp.concatenate([edge_index[0].astype(jnp.int32), loops,
                           jnp.zeros((npad_e,), jnp.int32)])
    dst = jnp.concatenate([edge_index[1].astype(jnp.int32), loops,
                           jnp.full((npad_e,), N, jnp.int32)])
    x_pad = jnp.pad(x, ((0, NPAD - N), (0, 0)))

    def split(h):
        return jnp.concatenate([h[:, q * DQ:(q + 1) * DQ] for q in range(4)],
                               axis=0)

    h1, sad1 = _tc_proj1(x_pad, W1, jnp.stack([a1_src, a1_dst], axis=1))
    den1, acc1 = _sc_edge(src, dst, sad1, split(h1))

    h2, sad2 = _tc_comb(acc1, den1, b1.reshape(1, D), W2,
                        jnp.stack([a2_src, a2_dst], axis=1))
    den2, acc2 = _sc_edge(src, dst, sad2, split(h2))

    h3, sad3 = _tc_comb(acc2, den2, b2.reshape(1, D), W3,
                        jnp.stack([a3_src, a3_dst], axis=1))
    den3, acc3 = _sc_edge(src, dst, sad3, split(h3))

    return _tc_final(acc3, den3, b3.reshape(1, D), x_pad, batch,
                     W_l1, b_l1.reshape(1, D), W_news, b_news.reshape(1, D),
                     W_l2[:D], W_l2[D:], b_l2.reshape(1, 1))


# speedup vs baseline: 32.6470x; 1.1553x over previous
import dataclasses
import functools

import jax
import jax.numpy as jnp
from jax import lax
from jax.experimental import pallas as pl
from jax.experimental.pallas import tpu as pltpu
from jax.experimental.pallas import tpu_sc as plsc

N = 10000
D = 128
G = 128
NPAD = 10240
NTILES = 32
EPAD = 331776
EPT = EPAD // 16
DH = 64
DQ = 32
CH = 128
RPT = NPAD // 16
ZR = 64
BLK = 1280

def _sc_edge_body(src_hbm, dst_hbm, sad_hbm, h_hbm, den_hbm, acc_hbm,
                  src_v, dst_v, as_v, ad_v, den_v, ex_v, rowb0, rowb1, rowb2,
                  dbuf0, dbuf1, dbuf2, gbuf0, gbuf1, gbuf2, zbuf, acc_sh,
                  sem0, sem1, sem2, ssem0, ssem1, ssem2):
    cid = lax.axis_index("c")
    sid = lax.axis_index("s")
    base = sid * EPT
    pltpu.sync_copy(src_hbm.at[pl.ds(base, EPT)], src_v)
    pltpu.sync_copy(dst_hbm.at[pl.ds(base, EPT)], dst_v)
    pltpu.sync_copy(sad_hbm.at[0], as_v)
    pltpu.sync_copy(sad_hbm.at[1], ad_v)

    zero16 = jnp.zeros((16,), jnp.float32)

    @pl.loop(0, NPAD, step=16)
    def _(i):
        den_v[pl.ds(i, 16)] = zero16

    @pl.loop(0, ZR)
    def _(r):
        for c in range(DQ // 16):
            zbuf[r, pl.ds(c * 16, 16)] = zero16

    @pl.loop(0, EPT, step=16)
    def _(i):
        s16 = src_v[pl.ds(i, 16)]
        d16 = dst_v[pl.ds(i, 16)]
        z = plsc.load_gather(as_v, [s16]) + plsc.load_gather(ad_v, [d16])
        e = jnp.maximum(z, 0.2 * z)
        ex = jnp.exp(e)
        ex_v[pl.ds(i, 16)] = ex
        plsc.addupdate_scatter(den_v, [d16], ex)

    @pl.when(cid == 0)
    def _():
        pltpu.sync_copy(den_v, den_hbm.at[sid])

    rowbs = (rowb0, rowb1, rowb2)
    dbufs = (dbuf0, dbuf1, dbuf2)
    gbufs = (gbuf0, gbuf1, gbuf2)
    sems = (sem0, sem1, sem2)
    ssems = (ssem0, ssem1, ssem2)

    for ph in range(2):
        q = cid * 2 + ph
        row_off = q * NPAD

        @pl.loop(0, RPT, step=ZR)
        def _(r):
            pltpu.sync_copy(zbuf, acc_sh.at[pl.ds(sid * RPT + r, ZR)])

        plsc.subcore_barrier()

        def wait_scatter(p):
            pltpu.make_async_copy(rowbs[p], acc_sh.at[dbufs[p]],
                                  ssems[p]).wait()

        def prep_and_start(i, p, pending_scatter):
            if pending_scatter:
                wait_scatter(p)
            for k in range(0, CH, 16):
                dbufs[p][pl.ds(k, 16)] = dst_v[pl.ds(i + k, 16)]
                gbufs[p][pl.ds(k, 16)] = src_v[pl.ds(i + k, 16)] + row_off
            pltpu.async_copy(h_hbm.at[gbufs[p]], rowbs[p], sems[p])

        def finish(i, p):
            pltpu.make_async_copy(h_hbm.at[gbufs[p]], rowbs[p],
                                  sems[p]).wait()

            @pl.loop(0, CH, step=16)
            def _(k):
                wv = ex_v[pl.ds(i + k, 16)]
                for e in range(16):
                    w = wv[e]
                    for c in range(DQ // 16):
                        sl = pl.ds(c * 16, 16)
                        rowbs[p][k + e, sl] = rowbs[p][k + e, sl] * w

            pltpu.async_copy(rowbs[p], acc_sh.at[dbufs[p]], add=True,
                             sem=ssems[p])

        prep_and_start(0, 0, False)
        prep_and_start(CH, 1, False)
        finish(0, 0)
        prep_and_start(2 * CH, 2, False)
        finish(CH, 1)
        prep_and_start(3 * CH, 0, True)
        finish(2 * CH, 2)

        @pl.loop(3 * CH, EPT - 3 * CH, step=3 * CH)
        def _(i):
            prep_and_start(i + CH, 1, True)
            finish(i, 0)
            prep_and_start(i + 2 * CH, 2, True)
            finish(i + CH, 1)
            prep_and_start(i + 3 * CH, 0, True)
            finish(i + 2 * CH, 2)

        prep_and_start(EPT - 2 * CH, 1, True)
        finish(EPT - 3 * CH, 0)
        prep_and_start(EPT - CH, 2, True)
        finish(EPT - 2 * CH, 1)
        finish(EPT - CH, 2)

        wait_scatter(0)
        wait_scatter(1)
        wait_scatter(2)

        plsc.subcore_barrier()

        @pl.loop(0, RPT, step=ZR)
        def _(r):
            pltpu.sync_copy(acc_sh.at[pl.ds(sid * RPT + r, ZR)],
                            acc_hbm.at[q].at[pl.ds(sid * RPT + r, ZR)])


def _sc_compiler_params():
    cp = pltpu.CompilerParams()
    fields = pltpu.CompilerParams.__dataclass_fields__
    if "needs_layout_passes" in fields:
        cp = dataclasses.replace(cp, needs_layout_passes=False)
    if "use_tc_tiling_on_sc" in fields:
        cp = dataclasses.replace(cp, use_tc_tiling_on_sc=False)
    return cp


def _sc_edge(src, dst, sad, h):
    k = pl.kernel(
        _sc_edge_body,
        out_type=(jax.ShapeDtypeStruct((16, NPAD), jnp.float32),
                  jax.ShapeDtypeStruct((4, NPAD, DQ), jnp.float32)),
        mesh=plsc.VectorSubcoreMesh(core_axis_name="c", subcore_axis_name="s"),
        scratch_types=[
            pltpu.VMEM((EPT,), jnp.int32),
            pltpu.VMEM((EPT,), jnp.int32),
            pltpu.VMEM((NPAD,), jnp.float32),
            pltpu.VMEM((NPAD,), jnp.float32),
            pltpu.VMEM((NPAD,), jnp.float32),
            pltpu.VMEM((EPT,), jnp.float32),
            pltpu.VMEM((CH, DQ), jnp.float32),
            pltpu.VMEM((CH, DQ), jnp.float32),
            pltpu.VMEM((CH, DQ), jnp.float32),
            pltpu.VMEM((CH,), jnp.int32),
            pltpu.VMEM((CH,), jnp.int32),
            pltpu.VMEM((CH,), jnp.int32),
            pltpu.VMEM((CH,), jnp.int32),
            pltpu.VMEM((CH,), jnp.int32),
            pltpu.VMEM((CH,), jnp.int32),
            pltpu.VMEM((ZR, DQ), jnp.float32),
            pltpu.VMEM_SHARED((NPAD, DQ), jnp.float32),
            pltpu.SemaphoreType.DMA,
            pltpu.SemaphoreType.DMA,
            pltpu.SemaphoreType.DMA,
            pltpu.SemaphoreType.DMA,
            pltpu.SemaphoreType.DMA,
            pltpu.SemaphoreType.DMA,
        ],
        compiler_params=_sc_compiler_params(),
    )
    return k(src, dst, sad, h)


def _tc_proj1_body(x_ref, w_ref, ap_ref, h_ref, sad_ref):
    h = jnp.dot(x_ref[...], w_ref[...], preferred_element_type=jnp.float32)
    h_ref[...] = h
    sad_ref[...] = lax.dot_general(ap_ref[...], h, (((0,), (1,)), ((), ())),
                                   preferred_element_type=jnp.float32)


def _tc_proj1(x_pad, w, ap):
    return pl.pallas_call(
        _tc_proj1_body,
        grid=(NPAD // BLK,),
        in_specs=[
            pl.BlockSpec((BLK, D), lambda i: (i, 0)),
            pl.BlockSpec((D, D), lambda i: (0, 0)),
            pl.BlockSpec((D, 2), lambda i: (0, 0)),
        ],
        out_specs=[
            pl.BlockSpec((BLK, D), lambda i: (i, 0)),
            pl.BlockSpec((2, BLK), lambda i: (0, i)),
        ],
        out_shape=[
            jax.ShapeDtypeStruct((NPAD, D), jnp.float32),
            jax.ShapeDtypeStruct((2, NPAD), jnp.float32),
        ],
    )(x_pad, w, ap)


def _tc_comb_body(acc_ref, den_ref, b_ref, w_ref, ap_ref, h_ref, sad_ref):
    i = pl.program_id(0)
    den = jnp.sum(den_ref[...], axis=0)
    invd = 1.0 / (den + 1e-16)
    rows = lax.broadcasted_iota(jnp.int32, (BLK, 1), 0) + i * BLK
    acc = jnp.concatenate([acc_ref[q] for q in range(4)], axis=1)
    hin = acc * invd[:, None] + b_ref[...]
    hin = jnp.where(rows < N, jnp.maximum(hin, 0.0), 0.0)
    h = jnp.dot(hin, w_ref[...], preferred_element_type=jnp.float32)
    h_ref[...] = h
    sad_ref[...] = lax.dot_general(ap_ref[...], h, (((0,), (1,)), ((), ())),
                                   preferred_element_type=jnp.float32)


def _tc_comb(acc, den, b, w, ap):
    return pl.pallas_call(
        _tc_comb_body,
        grid=(NPAD // BLK,),
        in_specs=[
            pl.BlockSpec((4, BLK, DQ), lambda i: (0, i, 0)),
            pl.BlockSpec((16, BLK), lambda i: (0, i)),
            pl.BlockSpec((1, D), lambda i: (0, 0)),
            pl.BlockSpec((D, D), lambda i: (0, 0)),
            pl.BlockSpec((D, 2), lambda i: (0, 0)),
        ],
        out_specs=[
            pl.BlockSpec((BLK, D), lambda i: (i, 0)),
            pl.BlockSpec((2, BLK), lambda i: (0, i)),
        ],
        out_shape=[
            jax.ShapeDtypeStruct((NPAD, D), jnp.float32),
            jax.ShapeDtypeStruct((2, NPAD), jnp.float32),
        ],
    )(acc, den, b, w, ap)


def _tc_final_body(acc_ref, den_ref, b3_ref, x_ref, batch_ref,
                   wl1_ref, bl1_ref, wn_ref, bn_ref, wa_ref, wb_ref, bl2_ref,
                   o_ref, h3_ref, gmax_ref, tbl_ref, xr_ref, rt_ref):
    den = jnp.sum(den_ref[...], axis=0)
    invd = 1.0 / (den + 1e-16)
    rows = lax.broadcasted_iota(jnp.int32, (NPAD, 1), 0)
    acc = jnp.concatenate([acc_ref[q] for q in range(4)], axis=1)
    h3 = acc * invd[:, None] + b3_ref[...]
    h3_ref[...] = jnp.where(rows < N, jnp.maximum(h3, 0.0), 0.0)
    tbl_ref[...] = jnp.full((G, D), -jnp.inf, jnp.float32)

    def init_rt(g, carry):
        rt_ref[g] = jnp.int32(2147483647)
        return carry

    lax.fori_loop(0, G, init_rt, 0)

    gmax_ref[...] = jnp.max(
        h3_ref[...].reshape(NPAD // 8, 8, D), axis=1)

    def pool_group(j, carry):
        r0 = j * 8
        g0 = batch_ref[r0]
        g7 = batch_ref[r0 + 7]

        def uniform(_):
            cur = tbl_ref[pl.ds(g0, 1), :]
            tbl_ref[pl.ds(g0, 1), :] = jnp.maximum(
                cur, gmax_ref[pl.ds(j, 1), :])
            rt_ref[g0] = jnp.minimum(rt_ref[g0], r0)
            return 0

        def mixed(_):
            def row_body(t, c):
                g = batch_ref[r0 + t]
                cur = tbl_ref[pl.ds(g, 1), :]
                tbl_ref[pl.ds(g, 1), :] = jnp.maximum(
                    cur, h3_ref[pl.ds(r0 + t, 1), :])
                rt_ref[g] = jnp.minimum(rt_ref[g], r0 + t)
                return c

            return lax.fori_loop(0, 8, row_body, 0)

        lax.cond(g0 == g7, uniform, mixed, 0)
        return carry

    lax.fori_loop(0, N // 8, pool_group, 0)

    pooled = tbl_ref[...]
    pooled = jnp.where(jnp.isfinite(pooled), pooled, 0.0)
    gm = jnp.maximum(
        jnp.dot(pooled, wl1_ref[...], preferred_element_type=jnp.float32)
        + bl1_ref[...], 0.0)

    def root_body(g, carry):
        idx = jnp.minimum(rt_ref[g], N - 1)
        xr_ref[pl.ds(g, 1), :] = x_ref[pl.ds(idx, 1), :]
        return carry

    lax.fori_loop(0, G, root_body, 0)

    news = jnp.maximum(
        jnp.dot(xr_ref[...], wn_ref[...], preferred_element_type=jnp.float32)
        + bn_ref[...], 0.0)
    logit = (jnp.dot(gm, wa_ref[...], preferred_element_type=jnp.float32)
             + jnp.dot(news, wb_ref[...], preferred_element_type=jnp.float32)
             + bl2_ref[...])
    o_ref[...] = jax.nn.sigmoid(logit)


def _tc_final(acc, den, b3, x_pad, batch, wl1, bl1, wn, bn, wa, wb, bl2):
    return pl.pallas_call(
        _tc_final_body,
        in_specs=[
            pl.BlockSpec((4, NPAD, DQ), lambda: (0, 0, 0)),
            pl.BlockSpec((16, NPAD), lambda: (0, 0)),
            pl.BlockSpec((1, D), lambda: (0, 0)),
            pl.BlockSpec((NPAD, D), lambda: (0, 0)),
            pl.BlockSpec(memory_space=pltpu.SMEM),
            pl.BlockSpec((D, D), lambda: (0, 0)),
            pl.BlockSpec((1, D), lambda: (0, 0)),
            pl.BlockSpec((D, D), lambda: (0, 0)),
            pl.BlockSpec((1, D), lambda: (0, 0)),
            pl.BlockSpec((D, 1), lambda: (0, 0)),
            pl.BlockSpec((D, 1), lambda: (0, 0)),
            pl.BlockSpec((1, 1), lambda: (0, 0)),
        ],
        out_specs=pl.BlockSpec((G, 1), lambda: (0, 0)),
        out_shape=jax.ShapeDtypeStruct((G, 1), jnp.float32),
        scratch_shapes=[
            pltpu.VMEM((NPAD, D), jnp.float32),
            pltpu.VMEM((NPAD // 8, D), jnp.float32),
            pltpu.VMEM((G, D), jnp.float32),
            pltpu.VMEM((G, D), jnp.float32),
            pltpu.SMEM((G,), jnp.int32),
        ],
    )(acc, den, b3, x_pad, batch, wl1, bl1, wn, bn, wa, wb, bl2)


def kernel(x, edge_index, batch, W1, a1_src, a1_dst, b1, W2, a2_src, a2_dst,
           b2, W3, a3_src, a3_dst, b3, W_news, b_news, W_l1, b_l1, W_l2,
           b_l2):
    e_real = edge_index.shape[1] + N
    npad_e = EPAD - e_real
    loops = jnp.arange(N, dtype=jnp.int32)
    src = jnp.concatenate([edge_index[0].astype(jnp.int32), loops,
                           jnp.zeros((npad_e,), jnp.int32)])
    dst = jnp.concatenate([edge_index[1].astype(jnp.int32), loops,
                           jnp.full((npad_e,), N, jnp.int32)])
    x_pad = jnp.pad(x, ((0, NPAD - N), (0, 0)))

    def split(h):
        return jnp.concatenate([h[:, q * DQ:(q + 1) * DQ] for q in range(4)],
                               axis=0)

    h1, sad1 = _tc_proj1(x_pad, W1, jnp.stack([a1_src, a1_dst], axis=1))
    den1, acc1 = _sc_edge(src, dst, sad1, split(h1))

    h2, sad2 = _tc_comb(acc1, den1, b1.reshape(1, D), W2,
                        jnp.stack([a2_src, a2_dst], axis=1))
    den2, acc2 = _sc_edge(src, dst, sad2, split(h2))

    h3, sad3 = _tc_comb(acc2, den2, b2.reshape(1, D), W3,
                        jnp.stack([a3_src, a3_dst], axis=1))
    den3, acc3 = _sc_edge(src, dst, sad3, split(h3))

    return _tc_final(acc3, den3, b3.reshape(1, D), x_pad, batch,
                     W_l1, b_l1.reshape(1, D), W_news, b_news.reshape(1, D),
                     W_l2[:D], W_l2[D:], b_l2.reshape(1, 1))


# speedup vs baseline: 35.9109x; 1.1000x over previous
import dataclasses
import functools

import jax
import jax.numpy as jnp
from jax import lax
from jax.experimental import pallas as pl
from jax.experimental.pallas import tpu as pltpu
from jax.experimental.pallas import tpu_sc as plsc

N = 10000
D = 128
G = 128
NPAD = 10240
NTILES = 32
EPAD = 331776
EPT = EPAD // 16
DH = 64
DQ = 32
CH = 128
RPT = NPAD // 16
ZR = 64
BLK = 1280

def _sc_edge_body(src_hbm, dst_hbm, sad_hbm, h_hbm, den_hbm, acc_hbm,
                  src_v, dst_v, as_v, ad_v, den_v, ex_v, rowb0, rowb1, rowb2,
                  dbuf0, dbuf1, dbuf2, gbuf0, gbuf1, gbuf2, zbuf, acc_sh,
                  sem0, sem1, sem2, ssem0, ssem1, ssem2):
    cid = lax.axis_index("c")
    sid = lax.axis_index("s")
    base = sid * EPT
    pltpu.sync_copy(src_hbm.at[pl.ds(base, EPT)], src_v)
    pltpu.sync_copy(dst_hbm.at[pl.ds(base, EPT)], dst_v)
    pltpu.sync_copy(sad_hbm.at[0], as_v)
    pltpu.sync_copy(sad_hbm.at[1], ad_v)

    zero16 = jnp.zeros((16,), jnp.float32)

    @pl.loop(0, NPAD, step=16)
    def _(i):
        den_v[pl.ds(i, 16)] = zero16

    @pl.loop(0, ZR)
    def _(r):
        for c in range(DQ // 16):
            zbuf[r, pl.ds(c * 16, 16)] = zero16

    @pl.loop(0, EPT, step=16)
    def _(i):
        s16 = src_v[pl.ds(i, 16)]
        d16 = dst_v[pl.ds(i, 16)]
        z = plsc.load_gather(as_v, [s16]) + plsc.load_gather(ad_v, [d16])
        e = jnp.maximum(z, 0.2 * z)
        ex = jnp.exp(e)
        ex_v[pl.ds(i, 16)] = ex
        plsc.addupdate_scatter(den_v, [d16], ex)

    @pl.when(cid == 0)
    def _():
        pltpu.sync_copy(den_v, den_hbm.at[sid])

    rowbs = (rowb0, rowb1, rowb2)
    dbufs = (dbuf0, dbuf1, dbuf2)
    gbufs = (gbuf0, gbuf1, gbuf2)
    sems = (sem0, sem1, sem2)
    ssems = (ssem0, ssem1, ssem2)

    for ph in range(2):
        q = cid * 2 + ph
        row_off = q * NPAD

        @pl.loop(0, RPT, step=ZR)
        def _(r):
            pltpu.sync_copy(zbuf, acc_sh.at[pl.ds(sid * RPT + r, ZR)])

        plsc.subcore_barrier()

        def wait_scatter(p):
            pltpu.make_async_copy(rowbs[p], acc_sh.at[dbufs[p]],
                                  ssems[p]).wait()

        def prep_and_start(i, p, pending_scatter):
            if pending_scatter:
                wait_scatter(p)
            for k in range(0, CH, 16):
                dbufs[p][pl.ds(k, 16)] = dst_v[pl.ds(i + k, 16)]
                gbufs[p][pl.ds(k, 16)] = src_v[pl.ds(i + k, 16)] + row_off
            pltpu.async_copy(h_hbm.at[gbufs[p]], rowbs[p], sems[p])

        def finish(i, p):
            pltpu.make_async_copy(h_hbm.at[gbufs[p]], rowbs[p],
                                  sems[p]).wait()

            @pl.loop(0, CH, step=16)
            def _(k):
                wv = ex_v[pl.ds(i + k, 16)]
                for e in range(16):
                    w = wv[e]
                    for c in range(DQ // 16):
                        sl = pl.ds(c * 16, 16)
                        rowbs[p][k + e, sl] = rowbs[p][k + e, sl] * w

            pltpu.async_copy(rowbs[p], acc_sh.at[dbufs[p]], add=True,
                             sem=ssems[p])

        prep_and_start(0, 0, False)
        prep_and_start(CH, 1, False)
        finish(0, 0)
        prep_and_start(2 * CH, 2, False)
        finish(CH, 1)
        prep_and_start(3 * CH, 0, True)
        finish(2 * CH, 2)

        @pl.loop(3 * CH, EPT - 3 * CH, step=3 * CH)
        def _(i):
            prep_and_start(i + CH, 1, True)
            finish(i, 0)
            prep_and_start(i + 2 * CH, 2, True)
            finish(i + CH, 1)
            prep_and_start(i + 3 * CH, 0, True)
            finish(i + 2 * CH, 2)

        prep_and_start(EPT - 2 * CH, 1, True)
        finish(EPT - 3 * CH, 0)
        prep_and_start(EPT - CH, 2, True)
        finish(EPT - 2 * CH, 1)
        finish(EPT - CH, 2)

        wait_scatter(0)
        wait_scatter(1)
        wait_scatter(2)

        plsc.subcore_barrier()

        @pl.loop(0, RPT, step=ZR)
        def _(r):
            pltpu.sync_copy(acc_sh.at[pl.ds(sid * RPT + r, ZR)],
                            acc_hbm.at[q].at[pl.ds(sid * RPT + r, ZR)])


def _sc_compiler_params():
    cp = pltpu.CompilerParams()
    fields = pltpu.CompilerParams.__dataclass_fields__
    if "needs_layout_passes" in fields:
        cp = dataclasses.replace(cp, needs_layout_passes=False)
    if "use_tc_tiling_on_sc" in fields:
        cp = dataclasses.replace(cp, use_tc_tiling_on_sc=False)
    return cp


def _sc_edge(src, dst, sad, h):
    k = pl.kernel(
        _sc_edge_body,
        out_type=(jax.ShapeDtypeStruct((16, NPAD), jnp.float32),
                  jax.ShapeDtypeStruct((4, NPAD, DQ), jnp.float32)),
        mesh=plsc.VectorSubcoreMesh(core_axis_name="c", subcore_axis_name="s"),
        scratch_types=[
            pltpu.VMEM((EPT,), jnp.int32),
            pltpu.VMEM((EPT,), jnp.int32),
            pltpu.VMEM((NPAD,), jnp.float32),
            pltpu.VMEM((NPAD,), jnp.float32),
            pltpu.VMEM((NPAD,), jnp.float32),
            pltpu.VMEM((EPT,), jnp.float32),
            pltpu.VMEM((CH, DQ), jnp.float32),
            pltpu.VMEM((CH, DQ), jnp.float32),
            pltpu.VMEM((CH, DQ), jnp.float32),
            pltpu.VMEM((CH,), jnp.int32),
            pltpu.VMEM((CH,), jnp.int32),
            pltpu.VMEM((CH,), jnp.int32),
            pltpu.VMEM((CH,), jnp.int32),
            pltpu.VMEM((CH,), jnp.int32),
            pltpu.VMEM((CH,), jnp.int32),
            pltpu.VMEM((ZR, DQ), jnp.float32),
            pltpu.VMEM_SHARED((NPAD, DQ), jnp.float32),
            pltpu.SemaphoreType.DMA,
            pltpu.SemaphoreType.DMA,
            pltpu.SemaphoreType.DMA,
            pltpu.SemaphoreType.DMA,
            pltpu.SemaphoreType.DMA,
            pltpu.SemaphoreType.DMA,
        ],
        compiler_params=_sc_compiler_params(),
    )
    return k(src, dst, sad, h)


def _store_quarters(h_ref, h):
    for q in range(4):
        h_ref[q] = h[:, q * DQ:(q + 1) * DQ]


def _tc_proj1_body(x_ref, w_ref, ap_ref, h_ref, sad_ref):
    h = jnp.dot(x_ref[...], w_ref[...], preferred_element_type=jnp.float32)
    _store_quarters(h_ref, h)
    sad_ref[...] = lax.dot_general(ap_ref[...], h, (((0,), (1,)), ((), ())),
                                   preferred_element_type=jnp.float32)


def _tc_proj1(x_pad, w, ap):
    return pl.pallas_call(
        _tc_proj1_body,
        grid=(NPAD // BLK,),
        in_specs=[
            pl.BlockSpec((BLK, D), lambda i: (i, 0)),
            pl.BlockSpec((D, D), lambda i: (0, 0)),
            pl.BlockSpec((D, 2), lambda i: (0, 0)),
        ],
        out_specs=[
            pl.BlockSpec((4, BLK, DQ), lambda i: (0, i, 0)),
            pl.BlockSpec((2, BLK), lambda i: (0, i)),
        ],
        out_shape=[
            jax.ShapeDtypeStruct((4, NPAD, DQ), jnp.float32),
            jax.ShapeDtypeStruct((2, NPAD), jnp.float32),
        ],
    )(x_pad, w, ap)


def _tc_comb_body(acc_ref, den_ref, b_ref, w_ref, ap_ref, h_ref, sad_ref):
    i = pl.program_id(0)
    den = jnp.sum(den_ref[...], axis=0)
    invd = 1.0 / (den + 1e-16)
    rows = lax.broadcasted_iota(jnp.int32, (BLK, 1), 0) + i * BLK
    acc = jnp.concatenate([acc_ref[q] for q in range(4)], axis=1)
    hin = acc * invd[:, None] + b_ref[...]
    hin = jnp.where(rows < N, jnp.maximum(hin, 0.0), 0.0)
    h = jnp.dot(hin, w_ref[...], preferred_element_type=jnp.float32)
    _store_quarters(h_ref, h)
    sad_ref[...] = lax.dot_general(ap_ref[...], h, (((0,), (1,)), ((), ())),
                                   preferred_element_type=jnp.float32)


def _tc_comb(acc, den, b, w, ap):
    return pl.pallas_call(
        _tc_comb_body,
        grid=(NPAD // BLK,),
        in_specs=[
            pl.BlockSpec((4, BLK, DQ), lambda i: (0, i, 0)),
            pl.BlockSpec((16, BLK), lambda i: (0, i)),
            pl.BlockSpec((1, D), lambda i: (0, 0)),
            pl.BlockSpec((D, D), lambda i: (0, 0)),
            pl.BlockSpec((D, 2), lambda i: (0, 0)),
        ],
        out_specs=[
            pl.BlockSpec((4, BLK, DQ), lambda i: (0, i, 0)),
            pl.BlockSpec((2, BLK), lambda i: (0, i)),
        ],
        out_shape=[
            jax.ShapeDtypeStruct((4, NPAD, DQ), jnp.float32),
            jax.ShapeDtypeStruct((2, NPAD), jnp.float32),
        ],
    )(acc, den, b, w, ap)


def _tc_final_body(acc_ref, den_ref, b3_ref, x_ref, batch_ref,
                   wl1_ref, bl1_ref, wn_ref, bn_ref, wa_ref, wb_ref, bl2_ref,
                   o_ref, h3_ref, gmax_ref, tbl_ref, xr_ref, rt_ref):
    den = jnp.sum(den_ref[...], axis=0)
    invd = 1.0 / (den + 1e-16)
    rows = lax.broadcasted_iota(jnp.int32, (NPAD, 1), 0)
    acc = jnp.concatenate([acc_ref[q] for q in range(4)], axis=1)
    h3 = acc * invd[:, None] + b3_ref[...]
    h3_ref[...] = jnp.where(rows < N, jnp.maximum(h3, 0.0), 0.0)
    tbl_ref[...] = jnp.full((G, D), -jnp.inf, jnp.float32)

    def init_rt(g, carry):
        rt_ref[g] = jnp.int32(2147483647)
        return carry

    lax.fori_loop(0, G, init_rt, 0)

    gmax_ref[...] = jnp.max(
        h3_ref[...].reshape(NPAD // 8, 8, D), axis=1)

    def pool_group(j, carry):
        r0 = j * 8
        g0 = batch_ref[r0]
        g7 = batch_ref[r0 + 7]

        def uniform(_):
            cur = tbl_ref[pl.ds(g0, 1), :]
            tbl_ref[pl.ds(g0, 1), :] = jnp.maximum(
                cur, gmax_ref[pl.ds(j, 1), :])
            rt_ref[g0] = jnp.minimum(rt_ref[g0], r0)
            return 0

        def mixed(_):
            def row_body(t, c):
                g = batch_ref[r0 + t]
                cur = tbl_ref[pl.ds(g, 1), :]
                tbl_ref[pl.ds(g, 1), :] = jnp.maximum(
                    cur, h3_ref[pl.ds(r0 + t, 1), :])
                rt_ref[g] = jnp.minimum(rt_ref[g], r0 + t)
                return c

            return lax.fori_loop(0, 8, row_body, 0)

        lax.cond(g0 == g7, uniform, mixed, 0)
        return carry

    lax.fori_loop(0, N // 8, pool_group, 0)

    pooled = tbl_ref[...]
    pooled = jnp.where(jnp.isfinite(pooled), pooled, 0.0)
    gm = jnp.maximum(
        jnp.dot(pooled, wl1_ref[...], preferred_element_type=jnp.float32)
        + bl1_ref[...], 0.0)

    def root_body(g, carry):
        idx = jnp.minimum(rt_ref[g], N - 1)
        xr_ref[pl.ds(g, 1), :] = x_ref[pl.ds(idx, 1), :]
        return carry

    lax.fori_loop(0, G, root_body, 0)

    news = jnp.maximum(
        jnp.dot(xr_ref[...], wn_ref[...], preferred_element_type=jnp.float32)
        + bn_ref[...], 0.0)
    logit = (jnp.dot(gm, wa_ref[...], preferred_element_type=jnp.float32)
             + jnp.dot(news, wb_ref[...], preferred_element_type=jnp.float32)
             + bl2_ref[...])
    o_ref[...] = jax.nn.sigmoid(logit)


def _tc_final(acc, den, b3, x_pad, batch, wl1, bl1, wn, bn, wa, wb, bl2):
    return pl.pallas_call(
        _tc_final_body,
        in_specs=[
            pl.BlockSpec((4, NPAD, DQ), lambda: (0, 0, 0)),
            pl.BlockSpec((16, NPAD), lambda: (0, 0)),
            pl.BlockSpec((1, D), lambda: (0, 0)),
            pl.BlockSpec((NPAD, D), lambda: (0, 0)),
            pl.BlockSpec(memory_space=pltpu.SMEM),
            pl.BlockSpec((D, D), lambda: (0, 0)),
            pl.BlockSpec((1, D), lambda: (0, 0)),
            pl.BlockSpec((D, D), lambda: (0, 0)),
            pl.BlockSpec((1, D), lambda: (0, 0)),
            pl.BlockSpec((D, 1), lambda: (0, 0)),
            pl.BlockSpec((D, 1), lambda: (0, 0)),
            pl.BlockSpec((1, 1), lambda: (0, 0)),
        ],
        out_specs=pl.BlockSpec((G, 1), lambda: (0, 0)),
        out_shape=jax.ShapeDtypeStruct((G, 1), jnp.float32),
        scratch_shapes=[
            pltpu.VMEM((NPAD, D), jnp.float32),
            pltpu.VMEM((NPAD // 8, D), jnp.float32),
            pltpu.VMEM((G, D), jnp.float32),
            pltpu.VMEM((G, D), jnp.float32),
            pltpu.SMEM((G,), jnp.int32),
        ],
    )(acc, den, b3, x_pad, batch, wl1, bl1, wn, bn, wa, wb, bl2)


def kernel(x, edge_index, batch, W1, a1_src, a1_dst, b1, W2, a2_src, a2_dst,
           b2, W3, a3_src, a3_dst, b3, W_news, b_news, W_l1, b_l1, W_l2,
           b_l2):
    e_real = edge_index.shape[1] + N
    npad_e = EPAD - e_real
    loops = jnp.arange(N, dtype=jnp.int32)
    src = jnp.concatenate([edge_index[0].astype(jnp.int32), loops,
                           jnp.zeros((npad_e,), jnp.int32)])
    dst = jnp.concatenate([edge_index[1].astype(jnp.int32), loops,
                           jnp.full((npad_e,), N, jnp.int32)])
    x_pad = jnp.pad(x, ((0, NPAD - N), (0, 0)))

    def flat(h):
        return h.reshape(4 * NPAD, DQ)

    h1, sad1 = _tc_proj1(x_pad, W1, jnp.stack([a1_src, a1_dst], axis=1))
    den1, acc1 = _sc_edge(src, dst, sad1, flat(h1))

    h2, sad2 = _tc_comb(acc1, den1, b1.reshape(1, D), W2,
                        jnp.stack([a2_src, a2_dst], axis=1))
    den2, acc2 = _sc_edge(src, dst, sad2, flat(h2))

    h3, sad3 = _tc_comb(acc2, den2, b2.reshape(1, D), W3,
                        jnp.stack([a3_src, a3_dst], axis=1))
    den3, acc3 = _sc_edge(src, dst, sad3, flat(h3))

    return _tc_final(acc3, den3, b3.reshape(1, D), x_pad, batch,
                     W_l1, b_l1.reshape(1, D), W_news, b_news.reshape(1, D),
                     W_l2[:D], W_l2[D:], b_l2.reshape(1, 1))


# speedup vs baseline: 37.1494x; 1.0345x over previous
import dataclasses
import functools

import jax
import jax.numpy as jnp
from jax import lax
from jax.experimental import pallas as pl
from jax.experimental.pallas import tpu as pltpu
from jax.experimental.pallas import tpu_sc as plsc

N = 10000
D = 128
G = 128
NPAD = 10240
NTILES = 32
EPAD = 331776
EPT = EPAD // 16
DH = 64
DQ = 32
CH = 128
RPT = NPAD // 16
ZR = 64
BLK = 1280

def _sc_edge_body(src_hbm, dst_hbm, sad_hbm, h_hbm, den_hbm, acc_hbm,
                  src_v, dst_v, as_v, ad_v, den_v, ex_v, rowb0, rowb1, rowb2,
                  dbuf0, dbuf1, dbuf2, gbuf0, gbuf1, gbuf2, zbuf, acc_sh,
                  sem0, sem1, sem2, ssem0, ssem1, ssem2):
    cid = lax.axis_index("c")
    sid = lax.axis_index("s")
    base = sid * EPT
    pltpu.sync_copy(src_hbm.at[pl.ds(base, EPT)], src_v)
    pltpu.sync_copy(dst_hbm.at[pl.ds(base, EPT)], dst_v)
    pltpu.sync_copy(sad_hbm.at[0], as_v)
    pltpu.sync_copy(sad_hbm.at[1], ad_v)

    zero16 = jnp.zeros((16,), jnp.float32)

    @pl.loop(0, NPAD, step=16)
    def _(i):
        den_v[pl.ds(i, 16)] = zero16

    @pl.loop(0, ZR)
    def _(r):
        for c in range(DQ // 16):
            zbuf[r, pl.ds(c * 16, 16)] = zero16

    @pl.loop(0, EPT, step=16)
    def _(i):
        s16 = src_v[pl.ds(i, 16)]
        d16 = dst_v[pl.ds(i, 16)]
        z = plsc.load_gather(as_v, [s16]) + plsc.load_gather(ad_v, [d16])
        e = jnp.maximum(z, 0.2 * z)
        ex = jnp.exp(e)
        ex_v[pl.ds(i, 16)] = ex
        plsc.addupdate_scatter(den_v, [d16], ex)

    @pl.when(cid == 0)
    def _():
        pltpu.sync_copy(den_v, den_hbm.at[sid])

    rowbs = (rowb0, rowb1, rowb2)
    dbufs = (dbuf0, dbuf1, dbuf2)
    gbufs = (gbuf0, gbuf1, gbuf2)
    sems = (sem0, sem1, sem2)
    ssems = (ssem0, ssem1, ssem2)

    for ph in range(2):
        q = cid * 2 + ph
        row_off = q * NPAD

        @pl.loop(0, RPT, step=ZR)
        def _(r):
            pltpu.sync_copy(zbuf, acc_sh.at[pl.ds(sid * RPT + r, ZR)])

        plsc.subcore_barrier()

        def wait_scatter(p):
            pltpu.make_async_copy(rowbs[p], acc_sh.at[dbufs[p]],
                                  ssems[p]).wait()

        def prep_and_start(i, p, pending_scatter):
            if pending_scatter:
                wait_scatter(p)
            for k in range(0, CH, 16):
                dbufs[p][pl.ds(k, 16)] = dst_v[pl.ds(i + k, 16)]
                gbufs[p][pl.ds(k, 16)] = src_v[pl.ds(i + k, 16)] + row_off
            pltpu.async_copy(h_hbm.at[gbufs[p]], rowbs[p], sems[p])

        def finish(i, p):
            pltpu.make_async_copy(h_hbm.at[gbufs[p]], rowbs[p],
                                  sems[p]).wait()

            @pl.loop(0, CH, step=16)
            def _(k):
                wv = ex_v[pl.ds(i + k, 16)]
                for e in range(16):
                    w = wv[e]
                    for c in range(DQ // 16):
                        sl = pl.ds(c * 16, 16)
                        rowbs[p][k + e, sl] = rowbs[p][k + e, sl] * w

            pltpu.async_copy(rowbs[p], acc_sh.at[dbufs[p]], add=True,
                             sem=ssems[p])

        prep_and_start(0, 0, False)
        prep_and_start(CH, 1, False)
        prep_and_start(2 * CH, 2, False)
        finish(0, 0)
        prep_and_start(3 * CH, 0, True)
        finish(CH, 1)
        prep_and_start(4 * CH, 1, True)
        finish(2 * CH, 2)

        @pl.loop(3 * CH, EPT - 4 * CH, step=3 * CH)
        def _(i):
            prep_and_start(i + 2 * CH, 2, True)
            finish(i, 0)
            prep_and_start(i + 3 * CH, 0, True)
            finish(i + CH, 1)
            prep_and_start(i + 4 * CH, 1, True)
            finish(i + 2 * CH, 2)

        prep_and_start(EPT - CH, 2, True)
        finish(EPT - 3 * CH, 0)
        finish(EPT - 2 * CH, 1)
        finish(EPT - CH, 2)

        wait_scatter(0)
        wait_scatter(1)
        wait_scatter(2)

        plsc.subcore_barrier()

        @pl.loop(0, RPT, step=ZR)
        def _(r):
            pltpu.sync_copy(acc_sh.at[pl.ds(sid * RPT + r, ZR)],
                            acc_hbm.at[q].at[pl.ds(sid * RPT + r, ZR)])


def _sc_compiler_params():
    cp = pltpu.CompilerParams()
    fields = pltpu.CompilerParams.__dataclass_fields__
    if "needs_layout_passes" in fields:
        cp = dataclasses.replace(cp, needs_layout_passes=False)
    if "use_tc_tiling_on_sc" in fields:
        cp = dataclasses.replace(cp, use_tc_tiling_on_sc=False)
    return cp


def _sc_edge(src, dst, sad, h):
    k = pl.kernel(
        _sc_edge_body,
        out_type=(jax.ShapeDtypeStruct((16, NPAD), jnp.float32),
                  jax.ShapeDtypeStruct((4, NPAD, DQ), jnp.float32)),
        mesh=plsc.VectorSubcoreMesh(core_axis_name="c", subcore_axis_name="s"),
        scratch_types=[
            pltpu.VMEM((EPT,), jnp.int32),
            pltpu.VMEM((EPT,), jnp.int32),
            pltpu.VMEM((NPAD,), jnp.float32),
            pltpu.VMEM((NPAD,), jnp.float32),
            pltpu.VMEM((NPAD,), jnp.float32),
            pltpu.VMEM((EPT,), jnp.float32),
            pltpu.VMEM((CH, DQ), jnp.float32),
            pltpu.VMEM((CH, DQ), jnp.float32),
            pltpu.VMEM((CH, DQ), jnp.float32),
            pltpu.VMEM((CH,), jnp.int32),
            pltpu.VMEM((CH,), jnp.int32),
            pltpu.VMEM((CH,), jnp.int32),
            pltpu.VMEM((CH,), jnp.int32),
            pltpu.VMEM((CH,), jnp.int32),
            pltpu.VMEM((CH,), jnp.int32),
            pltpu.VMEM((ZR, DQ), jnp.float32),
            pltpu.VMEM_SHARED((NPAD, DQ), jnp.float32),
            pltpu.SemaphoreType.DMA,
            pltpu.SemaphoreType.DMA,
            pltpu.SemaphoreType.DMA,
            pltpu.SemaphoreType.DMA,
            pltpu.SemaphoreType.DMA,
            pltpu.SemaphoreType.DMA,
        ],
        compiler_params=_sc_compiler_params(),
    )
    return k(src, dst, sad, h)


def _store_quarters(h_ref, h):
    for q in range(4):
        h_ref[q] = h[:, q * DQ:(q + 1) * DQ]


def _tc_proj1_body(x_ref, w_ref, ap_ref, h_ref, sad_ref):
    h = jnp.dot(x_ref[...], w_ref[...], preferred_element_type=jnp.float32)
    _store_quarters(h_ref, h)
    sad_ref[...] = lax.dot_general(ap_ref[...], h, (((0,), (1,)), ((), ())),
                                   preferred_element_type=jnp.float32)


def _tc_proj1(x_pad, w, ap):
    return pl.pallas_call(
        _tc_proj1_body,
        grid=(NPAD // BLK,),
        in_specs=[
            pl.BlockSpec((BLK, D), lambda i: (i, 0)),
            pl.BlockSpec((D, D), lambda i: (0, 0)),
            pl.BlockSpec((D, 2), lambda i: (0, 0)),
        ],
        out_specs=[
            pl.BlockSpec((4, BLK, DQ), lambda i: (0, i, 0)),
            pl.BlockSpec((2, BLK), lambda i: (0, i)),
        ],
        out_shape=[
            jax.ShapeDtypeStruct((4, NPAD, DQ), jnp.float32),
            jax.ShapeDtypeStruct((2, NPAD), jnp.float32),
        ],
    )(x_pad, w, ap)


def _tc_comb_body(acc_ref, den_ref, b_ref, w_ref, ap_ref, h_ref, sad_ref):
    i = pl.program_id(0)
    den = jnp.sum(den_ref[...], axis=0)
    invd = 1.0 / (den + 1e-16)
    rows = lax.broadcasted_iota(jnp.int32, (BLK, 1), 0) + i * BLK
    acc = jnp.concatenate([acc_ref[q] for q in range(4)], axis=1)
    hin = acc * invd[:, None] + b_ref[...]
    hin = jnp.where(rows < N, jnp.maximum(hin, 0.0), 0.0)
    h = jnp.dot(hin, w_ref[...], preferred_element_type=jnp.float32)
    _store_quarters(h_ref, h)
    sad_ref[...] = lax.dot_general(ap_ref[...], h, (((0,), (1,)), ((), ())),
                                   preferred_element_type=jnp.float32)


def _tc_comb(acc, den, b, w, ap):
    return pl.pallas_call(
        _tc_comb_body,
        grid=(NPAD // BLK,),
        in_specs=[
            pl.BlockSpec((4, BLK, DQ), lambda i: (0, i, 0)),
            pl.BlockSpec((16, BLK), lambda i: (0, i)),
            pl.BlockSpec((1, D), lambda i: (0, 0)),
            pl.BlockSpec((D, D), lambda i: (0, 0)),
            pl.BlockSpec((D, 2), lambda i: (0, 0)),
        ],
        out_specs=[
            pl.BlockSpec((4, BLK, DQ), lambda i: (0, i, 0)),
            pl.BlockSpec((2, BLK), lambda i: (0, i)),
        ],
        out_shape=[
            jax.ShapeDtypeStruct((4, NPAD, DQ), jnp.float32),
            jax.ShapeDtypeStruct((2, NPAD), jnp.float32),
        ],
    )(acc, den, b, w, ap)


def _tc_final_body(acc_ref, den_ref, b3_ref, x_ref, batch_ref,
                   wl1_ref, bl1_ref, wn_ref, bn_ref, wa_ref, wb_ref, bl2_ref,
                   o_ref, h3_ref, gmax_ref, tbl_ref, xr_ref, rt_ref):
    den = jnp.sum(den_ref[...], axis=0)
    invd = 1.0 / (den + 1e-16)
    rows = lax.broadcasted_iota(jnp.int32, (NPAD, 1), 0)
    acc = jnp.concatenate([acc_ref[q] for q in range(4)], axis=1)
    h3 = acc * invd[:, None] + b3_ref[...]
    h3_ref[...] = jnp.where(rows < N, jnp.maximum(h3, 0.0), 0.0)
    tbl_ref[...] = jnp.full((G, D), -jnp.inf, jnp.float32)

    def init_rt(g, carry):
        rt_ref[g] = jnp.int32(2147483647)
        return carry

    lax.fori_loop(0, G, init_rt, 0)

    gmax_ref[...] = jnp.max(
        h3_ref[...].reshape(NPAD // 8, 8, D), axis=1)

    def pool_group(j, carry):
        r0 = j * 8
        g0 = batch_ref[r0]
        g7 = batch_ref[r0 + 7]

        def uniform(_):
            cur = tbl_ref[pl.ds(g0, 1), :]
            tbl_ref[pl.ds(g0, 1), :] = jnp.maximum(
                cur, gmax_ref[pl.ds(j, 1), :])
            rt_ref[g0] = jnp.minimum(rt_ref[g0], r0)
            return 0

        def mixed(_):
            def row_body(t, c):
                g = batch_ref[r0 + t]
                cur = tbl_ref[pl.ds(g, 1), :]
                tbl_ref[pl.ds(g, 1), :] = jnp.maximum(
                    cur, h3_ref[pl.ds(r0 + t, 1), :])
                rt_ref[g] = jnp.minimum(rt_ref[g], r0 + t)
                return c

            return lax.fori_loop(0, 8, row_body, 0)

        lax.cond(g0 == g7, uniform, mixed, 0)
        return carry

    lax.fori_loop(0, N // 8, pool_group, 0)

    pooled = tbl_ref[...]
    pooled = jnp.where(jnp.isfinite(pooled), pooled, 0.0)
    gm = jnp.maximum(
        jnp.dot(pooled, wl1_ref[...], preferred_element_type=jnp.float32)
        + bl1_ref[...], 0.0)

    def root_body(g, carry):
        idx = jnp.minimum(rt_ref[g], N - 1)
        xr_ref[pl.ds(g, 1), :] = x_ref[pl.ds(idx, 1), :]
        return carry

    lax.fori_loop(0, G, root_body, 0)

    news = jnp.maximum(
        jnp.dot(xr_ref[...], wn_ref[...], preferred_element_type=jnp.float32)
        + bn_ref[...], 0.0)
    logit = (jnp.dot(gm, wa_ref[...], preferred_element_type=jnp.float32)
             + jnp.dot(news, wb_ref[...], preferred_element_type=jnp.float32)
             + bl2_ref[...])
    o_ref[...] = jax.nn.sigmoid(logit)


def _tc_final(acc, den, b3, x_pad, batch, wl1, bl1, wn, bn, wa, wb, bl2):
    return pl.pallas_call(
        _tc_final_body,
        in_specs=[
            pl.BlockSpec((4, NPAD, DQ), lambda: (0, 0, 0)),
            pl.BlockSpec((16, NPAD), lambda: (0, 0)),
            pl.BlockSpec((1, D), lambda: (0, 0)),
            pl.BlockSpec((NPAD, D), lambda: (0, 0)),
            pl.BlockSpec(memory_space=pltpu.SMEM),
            pl.BlockSpec((D, D), lambda: (0, 0)),
            pl.BlockSpec((1, D), lambda: (0, 0)),
            pl.BlockSpec((D, D), lambda: (0, 0)),
            pl.BlockSpec((1, D), lambda: (0, 0)),
            pl.BlockSpec((D, 1), lambda: (0, 0)),
            pl.BlockSpec((D, 1), lambda: (0, 0)),
            pl.BlockSpec((1, 1), lambda: (0, 0)),
        ],
        out_specs=pl.BlockSpec((G, 1), lambda: (0, 0)),
        out_shape=jax.ShapeDtypeStruct((G, 1), jnp.float32),
        scratch_shapes=[
            pltpu.VMEM((NPAD, D), jnp.float32),
            pltpu.VMEM((NPAD // 8, D), jnp.float32),
            pltpu.VMEM((G, D), jnp.float32),
            pltpu.VMEM((G, D), jnp.float32),
            pltpu.SMEM((G,), jnp.int32),
        ],
    )(acc, den, b3, x_pad, batch, wl1, bl1, wn, bn, wa, wb, bl2)


def kernel(x, edge_index, batch, W1, a1_src, a1_dst, b1, W2, a2_src, a2_dst,
           b2, W3, a3_src, a3_dst, b3, W_news, b_news, W_l1, b_l1, W_l2,
           b_l2):
    e_real = edge_index.shape[1] + N
    npad_e = EPAD - e_real
    loops = jnp.arange(N, dtype=jnp.int32)
    src = jnp.concatenate([edge_index[0].astype(jnp.int32), loops,
                           jnp.zeros((npad_e,), jnp.int32)])
    dst = jnp.concatenate([edge_index[1].astype(jnp.int32), loops,
                           jnp.full((npad_e,), N, jnp.int32)])
    x_pad = jnp.pad(x, ((0, NPAD - N), (0, 0)))

    def flat(h):
        return h.reshape(4 * NPAD, DQ)

    h1, sad1 = _tc_proj1(x_pad, W1, jnp.stack([a1_src, a1_dst], axis=1))
    den1, acc1 = _sc_edge(src, dst, sad1, flat(h1))

    h2, sad2 = _tc_comb(acc1, den1, b1.reshape(1, D), W2,
                        jnp.stack([a2_src, a2_dst], axis=1))
    den2, acc2 = _sc_edge(src, dst, sad2, flat(h2))

    h3, sad3 = _tc_comb(acc2, den2, b2.reshape(1, D), W3,
                        jnp.stack([a3_src, a3_dst], axis=1))
    den3, acc3 = _sc_edge(src, dst, sad3, flat(h3))

    return _tc_final(acc3, den3, b3.reshape(1, D), x_pad, batch,
                     W_l1, b_l1.reshape(1, D), W_news, b_news.reshape(1, D),
                     W_l2[:D], W_l2[D:], b_l2.reshape(1, 1))


# speedup vs baseline: 38.2402x; 1.0294x over previous
import dataclasses
import functools

import jax
import jax.numpy as jnp
from jax import lax
from jax.experimental import pallas as pl
from jax.experimental.pallas import tpu as pltpu
from jax.experimental.pallas import tpu_sc as plsc

N = 10000
D = 128
G = 128
NPAD = 10240
NTILES = 32
EPAD = 331776
EPT = EPAD // 16
DH = 64
DQ = 32
CH = 128
RPT = NPAD // 16
ZR = 64
BLK = 1280

def _sc_edge_body(src_hbm, dst_hbm, sad_hbm, h_hbm, den_hbm, acc_hbm,
                  src_v, dst_v, as_v, ad_v, den_v, ex_v, rowb0, rowb1, rowb2,
                  dbuf0, dbuf1, dbuf2, gbuf0, gbuf1, gbuf2, zbuf, acc_sh,
                  sem0, sem1, sem2, ssem0, ssem1, ssem2):
    cid = lax.axis_index("c")
    sid = lax.axis_index("s")
    base = sid * EPT
    pltpu.sync_copy(src_hbm.at[pl.ds(base, EPT)], src_v)
    pltpu.sync_copy(dst_hbm.at[pl.ds(base, EPT)], dst_v)
    pltpu.sync_copy(sad_hbm.at[0], as_v)
    pltpu.sync_copy(sad_hbm.at[1], ad_v)

    zero16 = jnp.zeros((16,), jnp.float32)

    @pl.loop(0, NPAD, step=16)
    def _(i):
        den_v[pl.ds(i, 16)] = zero16

    @pl.loop(0, ZR)
    def _(r):
        for c in range(DQ // 16):
            zbuf[r, pl.ds(c * 16, 16)] = zero16

    @pl.loop(0, EPT, step=16)
    def _(i):
        s16 = src_v[pl.ds(i, 16)]
        d16 = dst_v[pl.ds(i, 16)]
        z = plsc.load_gather(as_v, [s16]) + plsc.load_gather(ad_v, [d16])
        e = jnp.maximum(z, 0.2 * z)
        ex = jnp.exp(e)
        ex_v[pl.ds(i, 16)] = ex
        plsc.addupdate_scatter(den_v, [d16], ex)

    @pl.when(cid == 0)
    def _():
        pltpu.sync_copy(den_v, den_hbm.at[sid])

    rowbs = (rowb0, rowb1, rowb2)
    dbufs = (dbuf0, dbuf1, dbuf2)
    gbufs = (gbuf0, gbuf1, gbuf2)
    sems = (sem0, sem1, sem2)
    ssems = (ssem0, ssem1, ssem2)

    for ph in range(2):
        q = cid * 2 + ph
        row_off = q * NPAD

        @pl.loop(0, RPT, step=ZR)
        def _(r):
            pltpu.sync_copy(zbuf, acc_sh.at[pl.ds(sid * RPT + r, ZR)])

        plsc.subcore_barrier()

        def wait_scatter(p):
            pltpu.make_async_copy(rowbs[p], acc_sh.at[dbufs[p]],
                                  ssems[p]).wait()

        def prep_and_start(i, p, pending_scatter):
            if pending_scatter:
                wait_scatter(p)
            for k in range(0, CH, 16):
                dbufs[p][pl.ds(k, 16)] = dst_v[pl.ds(i + k, 16)]
                gbufs[p][pl.ds(k, 16)] = src_v[pl.ds(i + k, 16)] + row_off
            pltpu.async_copy(h_hbm.at[gbufs[p]], rowbs[p], sems[p])

        def finish(i, p):
            pltpu.make_async_copy(h_hbm.at[gbufs[p]], rowbs[p],
                                  sems[p]).wait()

            @plsc.parallel_loop(0, CH, step=16, unroll=2)
            def _(k):
                wv = ex_v[pl.ds(i + k, 16)]
                for e in range(16):
                    w = wv[e]
                    for c in range(DQ // 16):
                        sl = pl.ds(c * 16, 16)
                        rowbs[p][k + e, sl] = rowbs[p][k + e, sl] * w

            pltpu.async_copy(rowbs[p], acc_sh.at[dbufs[p]], add=True,
                             sem=ssems[p])

        prep_and_start(0, 0, False)
        prep_and_start(CH, 1, False)
        prep_and_start(2 * CH, 2, False)
        finish(0, 0)
        prep_and_start(3 * CH, 0, True)
        finish(CH, 1)
        prep_and_start(4 * CH, 1, True)
        finish(2 * CH, 2)

        @pl.loop(3 * CH, EPT - 4 * CH, step=3 * CH)
        def _(i):
            prep_and_start(i + 2 * CH, 2, True)
            finish(i, 0)
            prep_and_start(i + 3 * CH, 0, True)
            finish(i + CH, 1)
            prep_and_start(i + 4 * CH, 1, True)
            finish(i + 2 * CH, 2)

        prep_and_start(EPT - CH, 2, True)
        finish(EPT - 3 * CH, 0)
        finish(EPT - 2 * CH, 1)
        finish(EPT - CH, 2)

        wait_scatter(0)
        wait_scatter(1)
        wait_scatter(2)

        plsc.subcore_barrier()

        @pl.loop(0, RPT, step=ZR)
        def _(r):
            pltpu.sync_copy(acc_sh.at[pl.ds(sid * RPT + r, ZR)],
                            acc_hbm.at[q].at[pl.ds(sid * RPT + r, ZR)])


def _sc_compiler_params():
    cp = pltpu.CompilerParams()
    fields = pltpu.CompilerParams.__dataclass_fields__
    if "needs_layout_passes" in fields:
        cp = dataclasses.replace(cp, needs_layout_passes=False)
    if "use_tc_tiling_on_sc" in fields:
        cp = dataclasses.replace(cp, use_tc_tiling_on_sc=False)
    return cp


def _sc_edge(src, dst, sad, h):
    k = pl.kernel(
        _sc_edge_body,
        out_type=(jax.ShapeDtypeStruct((16, NPAD), jnp.float32),
                  jax.ShapeDtypeStruct((4, NPAD, DQ), jnp.float32)),
        mesh=plsc.VectorSubcoreMesh(core_axis_name="c", subcore_axis_name="s"),
        scratch_types=[
            pltpu.VMEM((EPT,), jnp.int32),
            pltpu.VMEM((EPT,), jnp.int32),
            pltpu.VMEM((NPAD,), jnp.float32),
            pltpu.VMEM((NPAD,), jnp.float32),
            pltpu.VMEM((NPAD,), jnp.float32),
            pltpu.VMEM((EPT,), jnp.float32),
            pltpu.VMEM((CH, DQ), jnp.float32),
            pltpu.VMEM((CH, DQ), jnp.float32),
            pltpu.VMEM((CH, DQ), jnp.float32),
            pltpu.VMEM((CH,), jnp.int32),
            pltpu.VMEM((CH,), jnp.int32),
            pltpu.VMEM((CH,), jnp.int32),
            pltpu.VMEM((CH,), jnp.int32),
            pltpu.VMEM((CH,), jnp.int32),
            pltpu.VMEM((CH,), jnp.int32),
            pltpu.VMEM((ZR, DQ), jnp.float32),
            pltpu.VMEM_SHARED((NPAD, DQ), jnp.float32),
            pltpu.SemaphoreType.DMA,
            pltpu.SemaphoreType.DMA,
            pltpu.SemaphoreType.DMA,
            pltpu.SemaphoreType.DMA,
            pltpu.SemaphoreType.DMA,
            pltpu.SemaphoreType.DMA,
        ],
        compiler_params=_sc_compiler_params(),
    )
    return k(src, dst, sad, h)


def _store_quarters(h_ref, h):
    for q in range(4):
        h_ref[q] = h[:, q * DQ:(q + 1) * DQ]


def _tc_proj1_body(x_ref, w_ref, ap_ref, h_ref, sad_ref):
    h = jnp.dot(x_ref[...], w_ref[...], preferred_element_type=jnp.float32)
    _store_quarters(h_ref, h)
    sad_ref[...] = lax.dot_general(ap_ref[...], h, (((0,), (1,)), ((), ())),
                                   preferred_element_type=jnp.float32)


def _tc_proj1(x_pad, w, ap):
    return pl.pallas_call(
        _tc_proj1_body,
        grid=(NPAD // BLK,),
        in_specs=[
            pl.BlockSpec((BLK, D), lambda i: (i, 0)),
            pl.BlockSpec((D, D), lambda i: (0, 0)),
            pl.BlockSpec((D, 2), lambda i: (0, 0)),
        ],
        out_specs=[
            pl.BlockSpec((4, BLK, DQ), lambda i: (0, i, 0)),
            pl.BlockSpec((2, BLK), lambda i: (0, i)),
        ],
        out_shape=[
            jax.ShapeDtypeStruct((4, NPAD, DQ), jnp.float32),
            jax.ShapeDtypeStruct((2, NPAD), jnp.float32),
        ],
    )(x_pad, w, ap)


def _tc_comb_body(acc_ref, den_ref, b_ref, w_ref, ap_ref, h_ref, sad_ref):
    i = pl.program_id(0)
    den = jnp.sum(den_ref[...], axis=0)
    invd = 1.0 / (den + 1e-16)
    rows = lax.broadcasted_iota(jnp.int32, (BLK, 1), 0) + i * BLK
    acc = jnp.concatenate([acc_ref[q] for q in range(4)], axis=1)
    hin = acc * invd[:, None] + b_ref[...]
    hin = jnp.where(rows < N, jnp.maximum(hin, 0.0), 0.0)
    h = jnp.dot(hin, w_ref[...], preferred_element_type=jnp.float32)
    _store_quarters(h_ref, h)
    sad_ref[...] = lax.dot_general(ap_ref[...], h, (((0,), (1,)), ((), ())),
                                   preferred_element_type=jnp.float32)


def _tc_comb(acc, den, b, w, ap):
    return pl.pallas_call(
        _tc_comb_body,
        grid=(NPAD // BLK,),
        in_specs=[
            pl.BlockSpec((4, BLK, DQ), lambda i: (0, i, 0)),
            pl.BlockSpec((16, BLK), lambda i: (0, i)),
            pl.BlockSpec((1, D), lambda i: (0, 0)),
            pl.BlockSpec((D, D), lambda i: (0, 0)),
            pl.BlockSpec((D, 2), lambda i: (0, 0)),
        ],
        out_specs=[
            pl.BlockSpec((4, BLK, DQ), lambda i: (0, i, 0)),
            pl.BlockSpec((2, BLK), lambda i: (0, i)),
        ],
        out_shape=[
            jax.ShapeDtypeStruct((4, NPAD, DQ), jnp.float32),
            jax.ShapeDtypeStruct((2, NPAD), jnp.float32),
        ],
    )(acc, den, b, w, ap)


def _tc_final_body(acc_ref, den_ref, b3_ref, x_ref, batch_ref,
                   wl1_ref, bl1_ref, wn_ref, bn_ref, wa_ref, wb_ref, bl2_ref,
                   o_ref, h3_ref, gmax_ref, tbl_ref, xr_ref, rt_ref):
    den = jnp.sum(den_ref[...], axis=0)
    invd = 1.0 / (den + 1e-16)
    rows = lax.broadcasted_iota(jnp.int32, (NPAD, 1), 0)
    acc = jnp.concatenate([acc_ref[q] for q in range(4)], axis=1)
    h3 = acc * invd[:, None] + b3_ref[...]
    h3_ref[...] = jnp.where(rows < N, jnp.maximum(h3, 0.0), 0.0)
    tbl_ref[...] = jnp.full((G, D), -jnp.inf, jnp.float32)

    def init_rt(g, carry):
        rt_ref[g] = jnp.int32(2147483647)
        return carry

    lax.fori_loop(0, G, init_rt, 0)

    gmax_ref[...] = jnp.max(
        h3_ref[...].reshape(NPAD // 8, 8, D), axis=1)

    def pool_group(j, carry):
        r0 = j * 8
        g0 = batch_ref[r0]
        g7 = batch_ref[r0 + 7]

        def uniform(_):
            cur = tbl_ref[pl.ds(g0, 1), :]
            tbl_ref[pl.ds(g0, 1), :] = jnp.maximum(
                cur, gmax_ref[pl.ds(j, 1), :])
            rt_ref[g0] = jnp.minimum(rt_ref[g0], r0)
            return 0

        def mixed(_):
            def row_body(t, c):
                g = batch_ref[r0 + t]
                cur = tbl_ref[pl.ds(g, 1), :]
                tbl_ref[pl.ds(g, 1), :] = jnp.maximum(
                    cur, h3_ref[pl.ds(r0 + t, 1), :])
                rt_ref[g] = jnp.minimum(rt_ref[g], r0 + t)
                return c

            return lax.fori_loop(0, 8, row_body, 0)

        lax.cond(g0 == g7, uniform, mixed, 0)
        return carry

    lax.fori_loop(0, N // 8, pool_group, 0)

    pooled = tbl_ref[...]
    pooled = jnp.where(jnp.isfinite(pooled), pooled, 0.0)
    gm = jnp.maximum(
        jnp.dot(pooled, wl1_ref[...], preferred_element_type=jnp.float32)
        + bl1_ref[...], 0.0)

    def root_body(g, carry):
        idx = jnp.minimum(rt_ref[g], N - 1)
        xr_ref[pl.ds(g, 1), :] = x_ref[pl.ds(idx, 1), :]
        return carry

    lax.fori_loop(0, G, root_body, 0)

    news = jnp.maximum(
        jnp.dot(xr_ref[...], wn_ref[...], preferred_element_type=jnp.float32)
        + bn_ref[...], 0.0)
    logit = (jnp.dot(gm, wa_ref[...], preferred_element_type=jnp.float32)
             + jnp.dot(news, wb_ref[...], preferred_element_type=jnp.float32)
             + bl2_ref[...])
    o_ref[...] = jax.nn.sigmoid(logit)


def _tc_final(acc, den, b3, x_pad, batch, wl1, bl1, wn, bn, wa, wb, bl2):
    return pl.pallas_call(
        _tc_final_body,
        in_specs=[
            pl.BlockSpec((4, NPAD, DQ), lambda: (0, 0, 0)),
            pl.BlockSpec((16, NPAD), lambda: (0, 0)),
            pl.BlockSpec((1, D), lambda: (0, 0)),
            pl.BlockSpec((NPAD, D), lambda: (0, 0)),
            pl.BlockSpec(memory_space=pltpu.SMEM),
            pl.BlockSpec((D, D), lambda: (0, 0)),
            pl.BlockSpec((1, D), lambda: (0, 0)),
            pl.BlockSpec((D, D), lambda: (0, 0)),
            pl.BlockSpec((1, D), lambda: (0, 0)),
            pl.BlockSpec((D, 1), lambda: (0, 0)),
            pl.BlockSpec((D, 1), lambda: (0, 0)),
            pl.BlockSpec((1, 1), lambda: (0, 0)),
        ],
        out_specs=pl.BlockSpec((G, 1), lambda: (0, 0)),
        out_shape=jax.ShapeDtypeStruct((G, 1), jnp.float32),
        scratch_shapes=[
            pltpu.VMEM((NPAD, D), jnp.float32),
            pltpu.VMEM((NPAD // 8, D), jnp.float32),
            pltpu.VMEM((G, D), jnp.float32),
            pltpu.VMEM((G, D), jnp.float32),
            pltpu.SMEM((G,), jnp.int32),
        ],
    )(acc, den, b3, x_pad, batch, wl1, bl1, wn, bn, wa, wb, bl2)


def kernel(x, edge_index, batch, W1, a1_src, a1_dst, b1, W2, a2_src, a2_dst,
           b2, W3, a3_src, a3_dst, b3, W_news, b_news, W_l1, b_l1, W_l2,
           b_l2):
    e_real = edge_index.shape[1] + N
    npad_e = EPAD - e_real
    loops = jnp.arange(N, dtype=jnp.int32)
    src = jnp.concatenate([edge_index[0].astype(jnp.int32), loops,
                           jnp.zeros((npad_e,), jnp.int32)])
    dst = jnp.concatenate([edge_index[1].astype(jnp.int32), loops,
                           jnp.full((npad_e,), N, jnp.int32)])
    x_pad = jnp.pad(x, ((0, NPAD - N), (0, 0)))

    def flat(h):
        return h.reshape(4 * NPAD, DQ)

    h1, sad1 = _tc_proj1(x_pad, W1, jnp.stack([a1_src, a1_dst], axis=1))
    den1, acc1 = _sc_edge(src, dst, sad1, flat(h1))

    h2, sad2 = _tc_comb(acc1, den1, b1.reshape(1, D), W2,
                        jnp.stack([a2_src, a2_dst], axis=1))
    den2, acc2 = _sc_edge(src, dst, sad2, flat(h2))

    h3, sad3 = _tc_comb(acc2, den2, b2.reshape(1, D), W3,
                        jnp.stack([a3_src, a3_dst], axis=1))
    den3, acc3 = _sc_edge(src, dst, sad3, flat(h3))

    return _tc_final(acc3, den3, b3.reshape(1, D), x_pad, batch,
                     W_l1, b_l1.reshape(1, D), W_news, b_news.reshape(1, D),
                     W_l2[:D], W_l2[D:], b_l2.reshape(1, 1))


# speedup vs baseline: 41.0928x; 1.0746x over previous
import dataclasses
import functools

import jax
import jax.numpy as jnp
from jax import lax
from jax.experimental import pallas as pl
from jax.experimental.pallas import tpu as pltpu
from jax.experimental.pallas import tpu_sc as plsc

N = 10000
D = 128
G = 128
NPAD = 10240
NTILES = 32
EPAD = 331776
EPT = EPAD // 16
DH = 64
DQ = 32
CH = 128
RPT = NPAD // 16
ZR = 64
BLK = 1280

def _sc_edge_body(src_hbm, dst_hbm, sad_hbm, h_hbm, den_hbm, acc_hbm,
                  src_v, dst_v, as_v, ad_v, den_v, ex_v, rowb0, rowb1, rowb2,
                  dbuf0, dbuf1, dbuf2, gbuf0, gbuf1, gbuf2, zbuf, acc_sh,
                  sem0, sem1, sem2, ssem0, ssem1, ssem2):
    cid = lax.axis_index("c")
    sid = lax.axis_index("s")
    base = sid * EPT
    pltpu.sync_copy(src_hbm.at[pl.ds(base, EPT)], src_v)
    pltpu.sync_copy(dst_hbm.at[pl.ds(base, EPT)], dst_v)
    pltpu.sync_copy(sad_hbm.at[0], as_v)
    pltpu.sync_copy(sad_hbm.at[1], ad_v)

    zero16 = jnp.zeros((16,), jnp.float32)

    @pl.loop(0, NPAD, step=16)
    def _(i):
        den_v[pl.ds(i, 16)] = zero16

    @pl.loop(0, ZR)
    def _(r):
        for c in range(DQ // 16):
            zbuf[r, pl.ds(c * 16, 16)] = zero16

    @plsc.parallel_loop(0, EPT, step=16, unroll=2)
    def _(i):
        s16 = src_v[pl.ds(i, 16)]
        d16 = dst_v[pl.ds(i, 16)]
        z = plsc.load_gather(as_v, [s16]) + plsc.load_gather(ad_v, [d16])
        e = jnp.maximum(z, 0.2 * z)
        ex = jnp.exp(e)
        ex_v[pl.ds(i, 16)] = ex
        plsc.addupdate_scatter(den_v, [d16], ex)

    @pl.when(cid == 0)
    def _():
        pltpu.sync_copy(den_v, den_hbm.at[sid])

    rowbs = (rowb0, rowb1, rowb2)
    dbufs = (dbuf0, dbuf1, dbuf2)
    gbufs = (gbuf0, gbuf1, gbuf2)
    sems = (sem0, sem1, sem2)
    ssems = (ssem0, ssem1, ssem2)

    for ph in range(2):
        q = cid * 2 + ph
        row_off = q * NPAD

        @pl.loop(0, RPT, step=ZR)
        def _(r):
            pltpu.sync_copy(zbuf, acc_sh.at[pl.ds(sid * RPT + r, ZR)])

        plsc.subcore_barrier()

        def wait_scatter(p):
            pltpu.make_async_copy(rowbs[p], acc_sh.at[dbufs[p]],
                                  ssems[p]).wait()

        def prep_and_start(i, p, pending_scatter):
            if pending_scatter:
                wait_scatter(p)
            for k in range(0, CH, 16):
                dbufs[p][pl.ds(k, 16)] = dst_v[pl.ds(i + k, 16)]
                gbufs[p][pl.ds(k, 16)] = src_v[pl.ds(i + k, 16)] + row_off
            pltpu.async_copy(h_hbm.at[gbufs[p]], rowbs[p], sems[p])

        def finish(i, p):
            pltpu.make_async_copy(h_hbm.at[gbufs[p]], rowbs[p],
                                  sems[p]).wait()

            @plsc.parallel_loop(0, CH, step=16, unroll=4)
            def _(k):
                wv = ex_v[pl.ds(i + k, 16)]
                for e in range(16):
                    w = wv[e]
                    for c in range(DQ // 16):
                        sl = pl.ds(c * 16, 16)
                        rowbs[p][k + e, sl] = rowbs[p][k + e, sl] * w

            pltpu.async_copy(rowbs[p], acc_sh.at[dbufs[p]], add=True,
                             sem=ssems[p])

        prep_and_start(0, 0, False)
        prep_and_start(CH, 1, False)
        prep_and_start(2 * CH, 2, False)
        finish(0, 0)
        prep_and_start(3 * CH, 0, True)
        finish(CH, 1)
        prep_and_start(4 * CH, 1, True)
        finish(2 * CH, 2)

        @pl.loop(3 * CH, EPT - 4 * CH, step=3 * CH)
        def _(i):
            prep_and_start(i + 2 * CH, 2, True)
            finish(i, 0)
            prep_and_start(i + 3 * CH, 0, True)
            finish(i + CH, 1)
            prep_and_start(i + 4 * CH, 1, True)
            finish(i + 2 * CH, 2)

        prep_and_start(EPT - CH, 2, True)
        finish(EPT - 3 * CH, 0)
        finish(EPT - 2 * CH, 1)
        finish(EPT - CH, 2)

        wait_scatter(0)
        wait_scatter(1)
        wait_scatter(2)

        plsc.subcore_barrier()

        @pl.loop(0, RPT, step=ZR)
        def _(r):
            pltpu.sync_copy(acc_sh.at[pl.ds(sid * RPT + r, ZR)],
                            acc_hbm.at[q].at[pl.ds(sid * RPT + r, ZR)])


def _sc_compiler_params():
    cp = pltpu.CompilerParams()
    fields = pltpu.CompilerParams.__dataclass_fields__
    if "needs_layout_passes" in fields:
        cp = dataclasses.replace(cp, needs_layout_passes=False)
    if "use_tc_tiling_on_sc" in fields:
        cp = dataclasses.replace(cp, use_tc_tiling_on_sc=False)
    return cp


def _sc_edge(src, dst, sad, h):
    k = pl.kernel(
        _sc_edge_body,
        out_type=(jax.ShapeDtypeStruct((16, NPAD), jnp.float32),
                  jax.ShapeDtypeStruct((4, NPAD, DQ), jnp.float32)),
        mesh=plsc.VectorSubcoreMesh(core_axis_name="c", subcore_axis_name="s"),
        scratch_types=[
            pltpu.VMEM((EPT,), jnp.int32),
            pltpu.VMEM((EPT,), jnp.int32),
            pltpu.VMEM((NPAD,), jnp.float32),
            pltpu.VMEM((NPAD,), jnp.float32),
            pltpu.VMEM((NPAD,), jnp.float32),
            pltpu.VMEM((EPT,), jnp.float32),
            pltpu.VMEM((CH, DQ), jnp.float32),
            pltpu.VMEM((CH, DQ), jnp.float32),
            pltpu.VMEM((CH, DQ), jnp.float32),
            pltpu.VMEM((CH,), jnp.int32),
            pltpu.VMEM((CH,), jnp.int32),
            pltpu.VMEM((CH,), jnp.int32),
            pltpu.VMEM((CH,), jnp.int32),
            pltpu.VMEM((CH,), jnp.int32),
            pltpu.VMEM((CH,), jnp.int32),
            pltpu.VMEM((ZR, DQ), jnp.float32),
            pltpu.VMEM_SHARED((NPAD, DQ), jnp.float32),
            pltpu.SemaphoreType.DMA,
            pltpu.SemaphoreType.DMA,
            pltpu.SemaphoreType.DMA,
            pltpu.SemaphoreType.DMA,
            pltpu.SemaphoreType.DMA,
            pltpu.SemaphoreType.DMA,
        ],
        compiler_params=_sc_compiler_params(),
    )
    return k(src, dst, sad, h)


def _store_quarters(h_ref, h):
    for q in range(4):
        h_ref[q] = h[:, q * DQ:(q + 1) * DQ]


def _tc_proj1_body(x_ref, w_ref, ap_ref, h_ref, sad_ref):
    h = jnp.dot(x_ref[...], w_ref[...], preferred_element_type=jnp.float32)
    _store_quarters(h_ref, h)
    sad_ref[...] = lax.dot_general(ap_ref[...], h, (((0,), (1,)), ((), ())),
                                   preferred_element_type=jnp.float32)


def _tc_proj1(x_pad, w, ap):
    return pl.pallas_call(
        _tc_proj1_body,
        grid=(NPAD // BLK,),
        in_specs=[
            pl.BlockSpec((BLK, D), lambda i: (i, 0)),
            pl.BlockSpec((D, D), lambda i: (0, 0)),
            pl.BlockSpec((D, 2), lambda i: (0, 0)),
        ],
        out_specs=[
            pl.BlockSpec((4, BLK, DQ), lambda i: (0, i, 0)),
            pl.BlockSpec((2, BLK), lambda i: (0, i)),
        ],
        out_shape=[
            jax.ShapeDtypeStruct((4, NPAD, DQ), jnp.float32),
            jax.ShapeDtypeStruct((2, NPAD), jnp.float32),
        ],
    )(x_pad, w, ap)


def _tc_comb_body(acc_ref, den_ref, b_ref, w_ref, ap_ref, h_ref, sad_ref):
    i = pl.program_id(0)
    den = jnp.sum(den_ref[...], axis=0)
    invd = 1.0 / (den + 1e-16)
    rows = lax.broadcasted_iota(jnp.int32, (BLK, 1), 0) + i * BLK
    acc = jnp.concatenate([acc_ref[q] for q in range(4)], axis=1)
    hin = acc * invd[:, None] + b_ref[...]
    hin = jnp.where(rows < N, jnp.maximum(hin, 0.0), 0.0)
    h = jnp.dot(hin, w_ref[...], preferred_element_type=jnp.float32)
    _store_quarters(h_ref, h)
    sad_ref[...] = lax.dot_general(ap_ref[...], h, (((0,), (1,)), ((), ())),
                                   preferred_element_type=jnp.float32)


def _tc_comb(acc, den, b, w, ap):
    return pl.pallas_call(
        _tc_comb_body,
        grid=(NPAD // BLK,),
        in_specs=[
            pl.BlockSpec((4, BLK, DQ), lambda i: (0, i, 0)),
            pl.BlockSpec((16, BLK), lambda i: (0, i)),
            pl.BlockSpec((1, D), lambda i: (0, 0)),
            pl.BlockSpec((D, D), lambda i: (0, 0)),
            pl.BlockSpec((D, 2), lambda i: (0, 0)),
        ],
        out_specs=[
            pl.BlockSpec((4, BLK, DQ), lambda i: (0, i, 0)),
            pl.BlockSpec((2, BLK), lambda i: (0, i)),
        ],
        out_shape=[
            jax.ShapeDtypeStruct((4, NPAD, DQ), jnp.float32),
            jax.ShapeDtypeStruct((2, NPAD), jnp.float32),
        ],
    )(acc, den, b, w, ap)


def _tc_final_body(acc_ref, den_ref, b3_ref, x_ref, batch_ref,
                   wl1_ref, bl1_ref, wn_ref, bn_ref, wa_ref, wb_ref, bl2_ref,
                   o_ref, h3_ref, gmax_ref, tbl_ref, xr_ref, rt_ref):
    den = jnp.sum(den_ref[...], axis=0)
    invd = 1.0 / (den + 1e-16)
    rows = lax.broadcasted_iota(jnp.int32, (NPAD, 1), 0)
    acc = jnp.concatenate([acc_ref[q] for q in range(4)], axis=1)
    h3 = acc * invd[:, None] + b3_ref[...]
    h3_ref[...] = jnp.where(rows < N, jnp.maximum(h3, 0.0), 0.0)
    tbl_ref[...] = jnp.full((G, D), -jnp.inf, jnp.float32)

    def init_rt(g, carry):
        rt_ref[g] = jnp.int32(2147483647)
        return carry

    lax.fori_loop(0, G, init_rt, 0)

    gmax_ref[...] = jnp.max(
        h3_ref[...].reshape(NPAD // 8, 8, D), axis=1)

    def pool_group(j, carry):
        r0 = j * 8
        g0 = batch_ref[r0]
        g7 = batch_ref[r0 + 7]

        def uniform(_):
            cur = tbl_ref[pl.ds(g0, 1), :]
            tbl_ref[pl.ds(g0, 1), :] = jnp.maximum(
                cur, gmax_ref[pl.ds(j, 1), :])
            rt_ref[g0] = jnp.minimum(rt_ref[g0], r0)
            return 0

        def mixed(_):
            def row_body(t, c):
                g = batch_ref[r0 + t]
                cur = tbl_ref[pl.ds(g, 1), :]
                tbl_ref[pl.ds(g, 1), :] = jnp.maximum(
                    cur, h3_ref[pl.ds(r0 + t, 1), :])
                rt_ref[g] = jnp.minimum(rt_ref[g], r0 + t)
                return c

            return lax.fori_loop(0, 8, row_body, 0)

        lax.cond(g0 == g7, uniform, mixed, 0)
        return carry

    lax.fori_loop(0, N // 8, pool_group, 0)

    pooled = tbl_ref[...]
    pooled = jnp.where(jnp.isfinite(pooled), pooled, 0.0)
    gm = jnp.maximum(
        jnp.dot(pooled, wl1_ref[...], preferred_element_type=jnp.float32)
        + bl1_ref[...], 0.0)

    def root_body(g, carry):
        idx = jnp.minimum(rt_ref[g], N - 1)
        xr_ref[pl.ds(g, 1), :] = x_ref[pl.ds(idx, 1), :]
        return carry

    lax.fori_loop(0, G, root_body, 0)

    news = jnp.maximum(
        jnp.dot(xr_ref[...], wn_ref[...], preferred_element_type=jnp.float32)
        + bn_ref[...], 0.0)
    logit = (jnp.dot(gm, wa_ref[...], preferred_element_type=jnp.float32)
             + jnp.dot(news, wb_ref[...], preferred_element_type=jnp.float32)
             + bl2_ref[...])
    o_ref[...] = jax.nn.sigmoid(logit)


def _tc_final(acc, den, b3, x_pad, batch, wl1, bl1, wn, bn, wa, wb, bl2):
    return pl.pallas_call(
        _tc_final_body,
        in_specs=[
            pl.BlockSpec((4, NPAD, DQ), lambda: (0, 0, 0)),
            pl.BlockSpec((16, NPAD), lambda: (0, 0)),
            pl.BlockSpec((1, D), lambda: (0, 0)),
            pl.BlockSpec((NPAD, D), lambda: (0, 0)),
            pl.BlockSpec(memory_space=pltpu.SMEM),
            pl.BlockSpec((D, D), lambda: (0, 0)),
            pl.BlockSpec((1, D), lambda: (0, 0)),
            pl.BlockSpec((D, D), lambda: (0, 0)),
            pl.BlockSpec((1, D), lambda: (0, 0)),
            pl.BlockSpec((D, 1), lambda: (0, 0)),
            pl.BlockSpec((D, 1), lambda: (0, 0)),
            pl.BlockSpec((1, 1), lambda: (0, 0)),
        ],
        out_specs=pl.BlockSpec((G, 1), lambda: (0, 0)),
        out_shape=jax.ShapeDtypeStruct((G, 1), jnp.float32),
        scratch_shapes=[
            pltpu.VMEM((NPAD, D), jnp.float32),
            pltpu.VMEM((NPAD // 8, D), jnp.float32),
            pltpu.VMEM((G, D), jnp.float32),
            pltpu.VMEM((G, D), jnp.float32),
            pltpu.SMEM((G,), jnp.int32),
        ],
    )(acc, den, b3, x_pad, batch, wl1, bl1, wn, bn, wa, wb, bl2)


def kernel(x, edge_index, batch, W1, a1_src, a1_dst, b1, W2, a2_src, a2_dst,
           b2, W3, a3_src, a3_dst, b3, W_news, b_news, W_l1, b_l1, W_l2,
           b_l2):
    e_real = edge_index.shape[1] + N
    npad_e = EPAD - e_real
    loops = jnp.arange(N, dtype=jnp.int32)
    src = jnp.concatenate([edge_index[0].astype(jnp.int32), loops,
                           jnp.zeros((npad_e,), jnp.int32)])
    dst = jnp.concatenate([edge_index[1].astype(jnp.int32), loops,
                           jnp.full((npad_e,), N, jnp.int32)])
    x_pad = jnp.pad(x, ((0, NPAD - N), (0, 0)))

    def flat(h):
        return h.reshape(4 * NPAD, DQ)

    h1, sad1 = _tc_proj1(x_pad, W1, jnp.stack([a1_src, a1_dst], axis=1))
    den1, acc1 = _sc_edge(src, dst, sad1, flat(h1))

    h2, sad2 = _tc_comb(acc1, den1, b1.reshape(1, D), W2,
                        jnp.stack([a2_src, a2_dst], axis=1))
    den2, acc2 = _sc_edge(src, dst, sad2, flat(h2))

    h3, sad3 = _tc_comb(acc2, den2, b2.reshape(1, D), W3,
                        jnp.stack([a3_src, a3_dst], axis=1))
    den3, acc3 = _sc_edge(src, dst, sad3, flat(h3))

    return _tc_final(acc3, den3, b3.reshape(1, D), x_pad, batch,
                     W_l1, b_l1.reshape(1, D), W_news, b_news.reshape(1, D),
                     W_l2[:D], W_l2[D:], b_l2.reshape(1, 1))
